# baseline (speedup 1.0000x reference)
import functools

import jax
import jax.numpy as jnp
from jax.experimental import pallas as pl
from jax.experimental.pallas import tpu as pltpu

N = 100000
E = 3200000
D_IN = 11
D_HID = 128
D_OUT = 2

BN = 1000


def _dense1_body(x_ref, h1_ref, h2_ref, w0_ref, w1_ref, w2_ref, b_ref, o_ref):
    acc = jnp.dot(x_ref[...], w0_ref[...], preferred_element_type=jnp.float32)
    acc += jnp.dot(h1_ref[...], w1_ref[...], preferred_element_type=jnp.float32)
    acc += jnp.dot(h2_ref[...], w2_ref[...], preferred_element_type=jnp.float32)
    acc += b_ref[...][None, :]
    o_ref[...] = jnp.maximum(acc, 0.0)


def _dense2_body(x_ref, h1_ref, h2_ref, w0_ref, w1_ref, w2_ref, b_ref, o_ref):
    acc = jnp.dot(x_ref[...], w0_ref[...], preferred_element_type=jnp.float32)
    acc += jnp.dot(h1_ref[...], w1_ref[...], preferred_element_type=jnp.float32)
    acc += jnp.dot(h2_ref[...], w2_ref[...], preferred_element_type=jnp.float32)
    acc += b_ref[...][None, :]
    m = jnp.max(acc, axis=1, keepdims=True)
    lse = jnp.log(jnp.sum(jnp.exp(acc - m), axis=1, keepdims=True)) + m
    o_ref[...] = acc - lse


def _dense1(x, h1, h2, w0, w1, w2, b):
    grid = (N // BN,)
    in_spec = pl.BlockSpec((BN, D_IN), lambda i: (i, 0))
    w_spec = pl.BlockSpec((D_IN, D_HID), lambda i: (0, 0))
    return pl.pallas_call(
        _dense1_body,
        grid=grid,
        in_specs=[in_spec, in_spec, in_spec, w_spec, w_spec, w_spec,
                  pl.BlockSpec((D_HID,), lambda i: (0,))],
        out_specs=pl.BlockSpec((BN, D_HID), lambda i: (i, 0)),
        out_shape=jax.ShapeDtypeStruct((N, D_HID), jnp.float32),
    )(x, h1, h2, w0, w1, w2, b)


def _dense2(x, h1, h2, w0, w1, w2, b):
    grid = (N // BN,)
    in_spec = pl.BlockSpec((BN, D_HID), lambda i: (i, 0))
    w_spec = pl.BlockSpec((D_HID, D_OUT), lambda i: (0, 0))
    return pl.pallas_call(
        _dense2_body,
        grid=grid,
        in_specs=[in_spec, in_spec, in_spec, w_spec, w_spec, w_spec,
                  pl.BlockSpec((D_OUT,), lambda i: (0,))],
        out_specs=pl.BlockSpec((BN, D_OUT), lambda i: (i, 0)),
        out_shape=jax.ShapeDtypeStruct((N, D_OUT), jnp.float32),
    )(x, h1, h2, w0, w1, w2, b)


def _seg_min(vals, idx, n):
    out = jax.ops.segment_min(vals, idx, num_segments=n)
    big = jnp.finfo(vals.dtype).max
    return jnp.where(out >= big, 0.0, out)


def kernel(x, edge_index, W1_0, W1_1, W1_2, b1, W2_0, W2_1, W2_2, b2):
    row = edge_index[0].astype(jnp.int32)
    col = edge_index[1].astype(jnp.int32)
    deg = jnp.zeros((N,), jnp.float32).at[col].add(1.0)
    dis = jnp.where(deg > 0, jax.lax.rsqrt(deg), 0.0)
    norm = dis[row] * dis[col]

    h11 = _seg_min(x[row] * norm[:, None], col, N)
    h12 = _seg_min(h11[row] * norm[:, None], col, N)
    h = _dense1(x, h11, h12, W1_0, W1_1, W1_2, b1)

    h21 = _seg_min(h[row] * norm[:, None], col, N)
    h22 = _seg_min(h21[row] * norm[:, None], col, N)
    return _dense2(h, h21, h22, W2_0, W2_1, W2_2, b2)


# speedup vs baseline: 5.5970x; 5.5970x over previous
import functools

import jax
import jax.numpy as jnp
from jax import lax
from jax.experimental import pallas as pl
from jax.experimental.pallas import tpu as pltpu
from jax.experimental.pallas import tpu_sc as plsc

N = 100000
E = 3200000
D_IN = 11
D_HID = 128
D_OUT = 2

NW = 32
RANGE = 3136
N_P = NW * RANGE
CAP = 102400
STG = 2048
CH = 4000
CH_D = 2048
ACC_ROWS = 3200
SENT_CL = RANGE
ECH = 128

BN = 1024

_mesh = plsc.VectorSubcoreMesh(
    core_axis_name="c", subcore_axis_name="s", num_cores=2, num_subcores=16)


def _wid():
    return lax.axis_index("s") * 2 + lax.axis_index("c")


PACK = 4096
INVALID = 0x7FFFFFFF


@functools.partial(
    pl.kernel,
    out_type=(
        jax.ShapeDtypeStruct((NW * CAP,), jnp.int32),
        jax.ShapeDtypeStruct((N_P,), jnp.float32),
    ),
    mesh=_mesh,
    compiler_params=pltpu.CompilerParams(
        needs_layout_passes=False, use_tc_tiling_on_sc=False),
    scratch_types=(
        pltpu.VMEM((CH,), jnp.int32),
        pltpu.VMEM((CH,), jnp.int32),
        pltpu.VMEM((STG + 16,), jnp.int32),
        pltpu.VMEM((ACC_ROWS + 16,), jnp.float32),
    ),
)
def _phase_a(row_hbm, col_hbm, eown, deg_out,
             colbuf, rowbuf, stage, degbuf):
    wid = _wid()
    c0 = wid * RANGE
    iota = lax.broadcasted_iota(jnp.int32, (16,), 0)
    zero16 = jnp.zeros((16,), jnp.float32)
    one0 = jnp.where(iota == 0, 1.0, 0.0).astype(jnp.float32)

    def sent16(base):
        return ((iota * 521 + base + wid * 1031) & 32767) * PACK + SENT_CL

    def zdeg(j, _):
        degbuf[pl.ds(j * 16, 16)] = zero16
        return 0
    lax.fori_loop(0, (ACC_ROWS + 16) // 16, zdeg, 0)

    def chunk(i, carry):
        pltpu.sync_copy(col_hbm.at[pl.ds(pl.multiple_of(i * CH, 8), CH)], colbuf)
        pltpu.sync_copy(row_hbm.at[pl.ds(pl.multiple_of(i * CH, 8), CH)], rowbuf)

        def vec(j, c2):
            pm, hp = c2
            cv = colbuf[pl.ds(j * 16, 16)]
            rv = rowbuf[pl.ds(j * 16, 16)]
            lv = cv - c0
            m = (lv >= 0) & (lv < RANGE)
            packed = jnp.where(m, rv * PACK + lv, jnp.int32(INVALID))
            packed = jnp.sort(packed)
            stage[pl.ds(pm, 16)] = packed
            pm = pm + plsc.all_reduce_population_count(m)[0]
            do_flush = (pm >= STG) & (hp + STG <= CAP)

            @pl.when(do_flush)
            def _():
                pltpu.sync_copy(stage.at[pl.ds(0, STG)],
                                eown.at[pl.ds(pl.multiple_of(wid * CAP + hp, STG), STG)])
                spill = stage[pl.ds(STG, 16)]
                stage[pl.ds(0, 16)] = spill

            hp = jnp.where(do_flush, hp + STG, hp)
            pm = jnp.where(do_flush, pm - STG, pm)
            return (pm, hp)

        return lax.fori_loop(0, CH // 16, vec, carry)

    pm, hp = lax.fori_loop(0, E // CH, chunk, (0, 0))

    def fin(j, _):
        base = j * 16
        keep = (iota + base) < pm
        cur = stage[pl.ds(base, 16)]
        stage[pl.ds(base, 16)] = jnp.where(keep, cur, sent16(base))
        return 0
    lax.fori_loop(0, STG // 16, fin, 0)

    def fill(i, hp2):
        @pl.when(hp2 < CAP)
        def _():
            pltpu.sync_copy(stage.at[pl.ds(0, STG)],
                            eown.at[pl.ds(pl.multiple_of(wid * CAP + hp2, STG), STG)])

        @pl.when(i == 0)
        def _():
            def allsent(j, _):
                base = j * 16
                stage[pl.ds(base, 16)] = sent16(base)
                return 0
            lax.fori_loop(0, STG // 16, allsent, 0)

        return jnp.where(hp2 < CAP, hp2 + STG, hp2)
    lax.fori_loop(0, CAP // STG, fill, hp)

    def degchunk(i, _):
        pltpu.sync_copy(
            eown.at[pl.ds(pl.multiple_of(wid * CAP + i * CH_D, CH_D), CH_D)],
            colbuf.at[pl.ds(0, CH_D)])

        def grp(g, _):
            clv = colbuf[pl.ds(g * 16, 16)] & (PACK - 1)
            for ee in range(16):
                cl = clv[ee]
                w = degbuf[pl.ds(cl, 16)]
                degbuf[pl.ds(cl, 16)] = w + one0
            return 0
        lax.fori_loop(0, CH_D // 16, grp, 0)
        return 0
    lax.fori_loop(0, CAP // CH_D, degchunk, 0)

    pltpu.sync_copy(degbuf.at[pl.ds(0, RANGE)],
                    deg_out.at[pl.ds(pl.multiple_of(wid * RANGE, 64), RANGE)])


def _make_round(FB, PP, want_g, blocked_src):
    out_type = [jax.ShapeDtypeStruct((PP * N_P * FB,), jnp.float32)]
    if want_g:
        out_type.append(jax.ShapeDtypeStruct((PP * N_P * FB,), jnp.float32))

    @functools.partial(
        pl.kernel,
        out_type=tuple(out_type),
        mesh=_mesh,
        compiler_params=pltpu.CompilerParams(
            needs_layout_passes=False, use_tc_tiling_on_sc=False),
        scratch_types=(
            pltpu.VMEM((ECH,), jnp.int32),
            pltpu.VMEM((ECH,), jnp.int32),
            pltpu.VMEM((ECH, FB), jnp.float32),
            pltpu.VMEM((ACC_ROWS * FB,), jnp.float32),
            pltpu.VMEM((RANGE,), jnp.float32),
            pltpu.SemaphoreType.DMA,
        ),
    )
    def round_k(g_hbm, eown, dis_hbm, *rest):
        if want_g:
            h_out, g_out = rest[0], rest[1]
            scratch = rest[2:]
        else:
            h_out = rest[0]
            scratch = rest[1:]
        ebuf, idxbuf, msgbuf, accflat, disbuf, sem = scratch
        wid = _wid()
        c0 = wid * RANGE
        inf16 = jnp.full((16,), jnp.inf, jnp.float32)

        pltpu.sync_copy(dis_hbm.at[pl.ds(pl.multiple_of(c0, 64), RANGE)], disbuf)

        for pass_ in range(PP):
            def zacc(j, _):
                accflat[pl.ds(j * 16, 16)] = inf16
                return 0
            lax.fori_loop(0, ACC_ROWS * FB // 16, zacc, 0)

            def chunk(i, _):
                pltpu.sync_copy(
                    eown.at[pl.ds(pl.multiple_of(wid * CAP + i * ECH, ECH), ECH)], ebuf)

                def mkidx(v, _):
                    rv = lax.shift_right_logical(ebuf[pl.ds(v * 16, 16)], 12)
                    if blocked_src:
                        idxbuf[pl.ds(v * 16, 16)] = rv + pass_ * N_P
                    else:
                        idxbuf[pl.ds(v * 16, 16)] = rv * PP + pass_
                    return 0
                lax.fori_loop(0, ECH // 16, mkidx, 0)
                pltpu.async_copy(g_hbm.at[idxbuf], msgbuf, sem).wait()

                def grp(g, _):
                    clv = ebuf[pl.ds(g * 16, 16)] & (PACK - 1)
                    for ee in range(16):
                        base = clv[ee] * FB
                        e = g * 16 + ee
                        for v in range(FB // 16):
                            a = accflat[pl.ds(base + v * 16, 16)]
                            mv = msgbuf[e, pl.ds(v * 16, 16)]
                            accflat[pl.ds(base + v * 16, 16)] = jnp.minimum(a, mv)
                    return 0
                lax.fori_loop(0, ECH // 16, grp, 0)
                return 0
            lax.fori_loop(0, CAP // ECH, chunk, 0)

            def hrow(r16, _):
                dvec = disbuf[pl.ds(r16 * 16, 16)]
                for rr in range(16):
                    dv = dvec[rr]
                    off = (r16 * 16 + rr) * FB
                    for v in range(FB // 16):
                        a = accflat[pl.ds(off + v * 16, 16)]
                        a = jnp.where(a == jnp.inf, 0.0, a) * dv
                        accflat[pl.ds(off + v * 16, 16)] = a
                return 0
            lax.fori_loop(0, RANGE // 16, hrow, 0)
            pltpu.sync_copy(
                accflat.at[pl.ds(0, RANGE * FB)],
                h_out.at[pl.ds(pl.multiple_of(pass_ * N_P * FB + c0 * FB, 256), RANGE * FB)])

            if want_g:
                def grow(r16, _):
                    dvec = disbuf[pl.ds(r16 * 16, 16)]
                    for rr in range(16):
                        dv = dvec[rr]
                        off = (r16 * 16 + rr) * FB
                        for v in range(FB // 16):
                            a = accflat[pl.ds(off + v * 16, 16)]
                            accflat[pl.ds(off + v * 16, 16)] = a * dv
                    return 0
                lax.fori_loop(0, RANGE // 16, grow, 0)
                pltpu.sync_copy(
                    accflat.at[pl.ds(0, RANGE * FB)],
                    g_out.at[pl.ds(pl.multiple_of(pass_ * N_P * FB + c0 * FB, 256), RANGE * FB)])

    return round_k


_round16 = _make_round(16, 1, True, False)
_round16_h = _make_round(16, 1, False, False)
_round128 = _make_round(32, 4, True, False)
_round128_h = _make_round(32, 4, False, True)


def _prep_body(deg_ref, x_ref, dis_ref, gx_ref):
    deg = deg_ref[...]
    dis = jnp.where(deg > 0, lax.rsqrt(deg), 0.0)
    dis_ref[...] = dis
    gx_ref[...] = x_ref[...] * dis[:, None]


def _prep(deg, x_p):
    return pl.pallas_call(
        _prep_body,
        grid=(N_P // BN,),
        in_specs=[pl.BlockSpec((BN,), lambda i: (i,)),
                  pl.BlockSpec((BN, 16), lambda i: (i, 0))],
        out_specs=[pl.BlockSpec((BN,), lambda i: (i,)),
                   pl.BlockSpec((BN, 16), lambda i: (i, 0))],
        out_shape=[jax.ShapeDtypeStruct((N_P,), jnp.float32),
                   jax.ShapeDtypeStruct((N_P, 16), jnp.float32)],
    )(deg, x_p)


def _dense1_body(x_ref, h1_ref, h2_ref, dis_ref, w0_ref, w1_ref, w2_ref,
                 b_ref, h_ref, gh_ref):
    acc = jnp.dot(x_ref[...], w0_ref[...], preferred_element_type=jnp.float32)
    acc += jnp.dot(h1_ref[...], w1_ref[...], preferred_element_type=jnp.float32)
    acc += jnp.dot(h2_ref[...], w2_ref[...], preferred_element_type=jnp.float32)
    acc += b_ref[...][None, :]
    h = jnp.maximum(acc, 0.0)
    h_ref[...] = h
    gh_ref[...] = h * dis_ref[...][:, None]


def _dense1(x_p, h1, h2, dis, w0, w1, w2, b):
    in16 = pl.BlockSpec((BN, 16), lambda i: (i, 0))
    w_spec = pl.BlockSpec((16, D_HID), lambda i: (0, 0))
    return pl.pallas_call(
        _dense1_body,
        grid=(N_P // BN,),
        in_specs=[in16, in16, in16, pl.BlockSpec((BN,), lambda i: (i,)),
                  w_spec, w_spec, w_spec,
                  pl.BlockSpec((D_HID,), lambda i: (0,))],
        out_specs=[pl.BlockSpec((BN, D_HID), lambda i: (i, 0)),
                   pl.BlockSpec((BN, D_HID), lambda i: (i, 0))],
        out_shape=[jax.ShapeDtypeStruct((N_P, D_HID), jnp.float32),
                   jax.ShapeDtypeStruct((N_P, D_HID), jnp.float32)],
    )(x_p, h1, h2, dis, w0, w1, w2, b)


def _dense2_body(h_ref, *refs):
    h21 = refs[0:4]
    h22 = refs[4:8]
    w0_ref = refs[8]
    w21 = refs[9:13]
    w22 = refs[13:17]
    b_ref = refs[17]
    o_ref = refs[18]
    acc = jnp.dot(h_ref[...], w0_ref[...], preferred_element_type=jnp.float32)
    for p in range(4):
        acc += jnp.dot(h21[p][0], w21[p][...],
                       preferred_element_type=jnp.float32)
        acc += jnp.dot(h22[p][0], w22[p][...],
                       preferred_element_type=jnp.float32)
    acc += b_ref[...][None, :]
    m = jnp.max(acc, axis=1, keepdims=True)
    lse = jnp.log(jnp.sum(jnp.exp(acc - m), axis=1, keepdims=True)) + m
    o_ref[...] = acc - lse


def _dense2(h, h21, h22, w0, w21s, w22s, b):
    in_h = pl.BlockSpec((BN, D_HID), lambda i: (i, 0))

    def pspec(p):
        return pl.BlockSpec((1, BN, 32), lambda i, p=p: (p, i, 0))

    wp_spec = pl.BlockSpec((32, D_OUT), lambda i: (0, 0))
    return pl.pallas_call(
        _dense2_body,
        grid=(N_P // BN,),
        in_specs=([in_h] + [pspec(p) for p in range(4)] * 2
                  + [pl.BlockSpec((D_HID, D_OUT), lambda i: (0, 0))]
                  + [wp_spec] * 8
                  + [pl.BlockSpec((D_OUT,), lambda i: (0,))]),
        out_specs=pl.BlockSpec((BN, D_OUT), lambda i: (i, 0)),
        out_shape=jax.ShapeDtypeStruct((N_P, D_OUT), jnp.float32),
    )(h, h21, h21, h21, h21, h22, h22, h22, h22,
      w0, *w21s, *w22s, b)


def kernel(x, edge_index, W1_0, W1_1, W1_2, b1, W2_0, W2_1, W2_2, b2):
    row = edge_index[0].astype(jnp.int32)
    col = edge_index[1].astype(jnp.int32)
    x_p = jnp.pad(x, ((0, N_P - N), (0, 16 - D_IN)))
    w1_0 = jnp.pad(W1_0, ((0, 16 - D_IN), (0, 0)))
    w1_1 = jnp.pad(W1_1, ((0, 16 - D_IN), (0, 0)))
    w1_2 = jnp.pad(W1_2, ((0, 16 - D_IN), (0, 0)))
    w21s = [W2_1[32 * p:32 * (p + 1)] for p in range(4)]
    w22s = [W2_2[32 * p:32 * (p + 1)] for p in range(4)]

    eown, deg = _phase_a(row, col)
    dis, gx = _prep(deg, x_p)

    h11_f, g11_f = _round16(gx, eown, dis)
    h11 = h11_f.reshape(N_P, 16)
    g11 = g11_f.reshape(N_P, 16)
    (h12_f,) = _round16_h(g11, eown, dis)
    h12 = h12_f.reshape(N_P, 16)
    h, gh = _dense1(x_p, h11, h12, dis, w1_0, w1_1, w1_2, b1)

    h21_f, g21_f = _round128(gh.reshape(N_P * 4, 32), eown, dis)
    h21 = h21_f.reshape(4, N_P, 32)
    (h22_f,) = _round128_h(g21_f.reshape(N_P * 4, 32), eown, dis)
    h22 = h22_f.reshape(4, N_P, 32)

    out = _dense2(h, h21, h22, W2_0, w21s, w22s, b2)
    return out[:N]


# speedup vs baseline: 7.4295x; 1.3274x over previous
import functools

import jax
import jax.numpy as jnp
from jax import lax
from jax.experimental import pallas as pl
from jax.experimental.pallas import tpu as pltpu
from jax.experimental.pallas import tpu_sc as plsc

N = 100000
E = 3200000
D_IN = 11
D_HID = 128
D_OUT = 2

NW = 32
RANGE = 3136
N_P = NW * RANGE
CAP = 102400
STG = 2048
CH = 4000
CH_D = 2048
ACC_ROWS = 3200
SENT_CL = RANGE
ECH = 128

BN = 1024

_mesh = plsc.VectorSubcoreMesh(
    core_axis_name="c", subcore_axis_name="s", num_cores=2, num_subcores=16)


def _wid():
    return lax.axis_index("s") * 2 + lax.axis_index("c")


PACK = 4096
INVALID = 0x7FFFFFFF


@functools.partial(
    pl.kernel,
    out_type=(
        jax.ShapeDtypeStruct((NW * CAP,), jnp.int32),
        jax.ShapeDtypeStruct((N_P,), jnp.float32),
    ),
    mesh=_mesh,
    compiler_params=pltpu.CompilerParams(
        needs_layout_passes=False, use_tc_tiling_on_sc=False),
    scratch_types=(
        pltpu.VMEM((2 * CH,), jnp.int32),
        pltpu.VMEM((2 * CH,), jnp.int32),
        pltpu.VMEM((STG + 16,), jnp.int32),
        pltpu.VMEM((ACC_ROWS + 16,), jnp.float32),
        pltpu.SemaphoreType.DMA,
        pltpu.SemaphoreType.DMA,
    ),
)
def _phase_a(row_hbm, col_hbm, eown, deg_out,
             colbuf, rowbuf, stage, degbuf, asem0, asem1):
    wid = _wid()
    c0 = wid * RANGE
    iota = lax.broadcasted_iota(jnp.int32, (16,), 0)
    zero16 = jnp.zeros((16,), jnp.float32)
    one0 = jnp.where(iota == 0, 1.0, 0.0).astype(jnp.float32)
    asems = (asem0, asem1)

    def sent16(base):
        return ((iota * 521 + base + wid * 1031) & 32767) * PACK + SENT_CL

    def zdeg(j, _):
        degbuf[pl.ds(j * 16, 16)] = zero16
        return 0
    lax.fori_loop(0, (ACC_ROWS + 16) // 16, zdeg, 0)

    NCH = E // CH

    def load(i, par):
        off = par * CH
        pltpu.async_copy(col_hbm.at[pl.ds(pl.multiple_of(i * CH, 8), CH)],
                         colbuf.at[pl.ds(off, CH)], asems[par])
        pltpu.async_copy(row_hbm.at[pl.ds(pl.multiple_of(i * CH, 8), CH)],
                         rowbuf.at[pl.ds(off, CH)], asems[par])

    def drain(i, par):
        off = par * CH
        pltpu.make_async_copy(
            col_hbm.at[pl.ds(pl.multiple_of(i * CH, 8), CH)],
            colbuf.at[pl.ds(off, CH)], asems[par]).wait()
        pltpu.make_async_copy(
            row_hbm.at[pl.ds(pl.multiple_of(i * CH, 8), CH)],
            rowbuf.at[pl.ds(off, CH)], asems[par]).wait()

    load(0, 0)

    def chunk(i, carry):
        par_t = i & 1

        @pl.when((par_t == 0) & (i + 1 < NCH))
        def _():
            load(i + 1, 1)

        @pl.when((par_t == 1) & (i + 1 < NCH))
        def _():
            load(i + 1, 0)

        @pl.when(par_t == 0)
        def _():
            drain(i, 0)

        @pl.when(par_t == 1)
        def _():
            drain(i, 1)

        boff = par_t * CH

        def vec(j, c2):
            pm, hp = c2
            cv = colbuf[pl.ds(boff + j * 16, 16)]
            lv = cv - c0
            m = (lv >= 0) & (lv < RANGE)
            cnt = plsc.all_reduce_population_count(m)[0]

            @pl.when(cnt > 0)
            def _():
                rv = rowbuf[pl.ds(boff + j * 16, 16)]
                packed = jnp.where(m, rv * PACK + lv, jnp.int32(INVALID))
                packed = jnp.sort(packed)
                stage[pl.ds(pm, 16)] = packed

            pm = pm + cnt
            do_flush = (pm >= STG) & (hp + STG <= CAP)

            @pl.when(do_flush)
            def _():
                pltpu.sync_copy(
                    stage.at[pl.ds(0, STG)],
                    eown.at[pl.ds(pl.multiple_of(wid * CAP + hp, STG), STG)])
                spill = stage[pl.ds(STG, 16)]
                stage[pl.ds(0, 16)] = spill

            hp = jnp.where(do_flush, hp + STG, hp)
            pm = jnp.where(do_flush, pm - STG, pm)
            return (pm, hp)

        return lax.fori_loop(0, CH // 16, vec, carry)

    pm, hp = lax.fori_loop(0, NCH, chunk, (0, 0))

    def fin(j, _):
        base = j * 16
        keep = (iota + base) < pm
        cur = stage[pl.ds(base, 16)]
        stage[pl.ds(base, 16)] = jnp.where(keep, cur, sent16(base))
        return 0
    lax.fori_loop(0, STG // 16, fin, 0)

    def fill(i, hp2):
        @pl.when(hp2 < CAP)
        def _():
            pltpu.sync_copy(
                stage.at[pl.ds(0, STG)],
                eown.at[pl.ds(pl.multiple_of(wid * CAP + hp2, STG), STG)])

        @pl.when(i == 0)
        def _():
            def allsent(j, _):
                base = j * 16
                stage[pl.ds(base, 16)] = sent16(base)
                return 0
            lax.fori_loop(0, STG // 16, allsent, 0)

        return jnp.where(hp2 < CAP, hp2 + STG, hp2)
    lax.fori_loop(0, CAP // STG, fill, hp)

    def degchunk(i, _):
        pltpu.sync_copy(
            eown.at[pl.ds(pl.multiple_of(wid * CAP + i * CH_D, CH_D), CH_D)],
            colbuf.at[pl.ds(0, CH_D)])

        def grp(g, _):
            clv = colbuf[pl.ds(g * 16, 16)] & (PACK - 1)
            for ee in range(16):
                cl = clv[ee]
                w = degbuf[pl.ds(cl, 16)]
                degbuf[pl.ds(cl, 16)] = w + one0
            return 0
        lax.fori_loop(0, CH_D // 16, grp, 0)
        return 0
    lax.fori_loop(0, CAP // CH_D, degchunk, 0)

    pltpu.sync_copy(degbuf.at[pl.ds(0, RANGE)],
                    deg_out.at[pl.ds(pl.multiple_of(wid * RANGE, 64), RANGE)])


K_PIPE = 2
GRP = ECH * K_PIPE
NG = CAP // GRP
ACC_R = 3152


def _make_round(FB, PP, want_g, blocked_src):
    out_type = [jax.ShapeDtypeStruct((PP * N_P * FB,), jnp.float32)]
    if want_g:
        out_type.append(jax.ShapeDtypeStruct((PP * N_P * FB,), jnp.float32))

    @functools.partial(
        pl.kernel,
        out_type=tuple(out_type),
        mesh=_mesh,
        compiler_params=pltpu.CompilerParams(
            needs_layout_passes=False, use_tc_tiling_on_sc=False),
        scratch_types=(
            pltpu.VMEM((2 * GRP,), jnp.int32),
            pltpu.VMEM((2 * GRP,), jnp.int32),
            pltpu.VMEM((2 * GRP, FB), jnp.float32),
            pltpu.VMEM((ACC_R * FB,), jnp.float32),
            pltpu.VMEM((RANGE,), jnp.float32),
            pltpu.SemaphoreType.DMA,
            pltpu.SemaphoreType.DMA,
        ),
    )
    def round_k(g_hbm, eown, dis_hbm, *rest):
        if want_g:
            h_out, g_out = rest[0], rest[1]
            scratch = rest[2:]
        else:
            h_out = rest[0]
            scratch = rest[1:]
        ebuf, idxbuf, msgbuf, accflat, disbuf, gsem0, gsem1 = scratch
        gsems = (gsem0, gsem1)
        wid = _wid()
        c0 = wid * RANGE
        inf16 = jnp.full((16,), jnp.inf, jnp.float32)

        pltpu.sync_copy(dis_hbm.at[pl.ds(pl.multiple_of(c0, 64), RANGE)],
                        disbuf)

        for pass_ in range(PP):
            def fire(gi, par):
                off = par * GRP
                pltpu.sync_copy(
                    eown.at[pl.ds(pl.multiple_of(wid * CAP + gi * GRP, GRP),
                                  GRP)],
                    ebuf.at[pl.ds(off, GRP)])

                def mkidx(v, _):
                    rv = lax.shift_right_logical(
                        ebuf[pl.ds(off + v * 16, 16)], 12)
                    if blocked_src:
                        idxbuf[pl.ds(off + v * 16, 16)] = rv + pass_ * N_P
                    else:
                        idxbuf[pl.ds(off + v * 16, 16)] = rv * PP + pass_
                    return 0
                lax.fori_loop(0, GRP // 16, mkidx, 0)
                for k in range(K_PIPE):
                    pltpu.async_copy(
                        g_hbm.at[idxbuf.at[pl.ds(off + k * ECH, ECH)]],
                        msgbuf.at[pl.ds(off + k * ECH, ECH)],
                        gsems[par])

            def drain_process(par):
                off = par * GRP
                for k in range(K_PIPE):
                    pltpu.make_async_copy(
                        g_hbm.at[idxbuf.at[pl.ds(off + k * ECH, ECH)]],
                        msgbuf.at[pl.ds(off + k * ECH, ECH)],
                        gsems[par]).wait()

                def grp(g, _):
                    clv = ebuf[pl.ds(off + g * 16, 16)] & (PACK - 1)
                    for ee in range(16):
                        base = clv[ee] * FB
                        r = off + g * 16 + ee
                        for v in range(FB // 16):
                            a = accflat[pl.ds(base + v * 16, 16)]
                            mv = msgbuf[r, pl.ds(v * 16, 16)]
                            accflat[pl.ds(base + v * 16, 16)] = \
                                jnp.minimum(a, mv)
                    return 0
                lax.fori_loop(0, GRP // 16, grp, 0)

            def zacc(j, _):
                accflat[pl.ds(j * 16, 16)] = inf16
                return 0
            lax.fori_loop(0, ACC_R * FB // 16, zacc, 0)

            fire(0, 0)

            def body(g, _):
                par_t = g & 1

                @pl.when((par_t == 0) & (g + 1 < NG))
                def _():
                    fire(g + 1, 1)

                @pl.when((par_t == 1) & (g + 1 < NG))
                def _():
                    fire(g + 1, 0)

                @pl.when(par_t == 0)
                def _():
                    drain_process(0)

                @pl.when(par_t == 1)
                def _():
                    drain_process(1)
                return 0
            lax.fori_loop(0, NG, body, 0)

            def hrow(r16, _):
                dvec = disbuf[pl.ds(r16 * 16, 16)]
                for rr in range(16):
                    dv = dvec[rr]
                    off = (r16 * 16 + rr) * FB
                    for v in range(FB // 16):
                        a = accflat[pl.ds(off + v * 16, 16)]
                        a = jnp.where(a == jnp.inf, 0.0, a) * dv
                        accflat[pl.ds(off + v * 16, 16)] = a
                return 0
            lax.fori_loop(0, RANGE // 16, hrow, 0)
            pltpu.sync_copy(
                accflat.at[pl.ds(0, RANGE * FB)],
                h_out.at[pl.ds(pl.multiple_of(
                    pass_ * N_P * FB + c0 * FB, 256), RANGE * FB)])

            if want_g:
                def grow(r16, _):
                    dvec = disbuf[pl.ds(r16 * 16, 16)]
                    for rr in range(16):
                        dv = dvec[rr]
                        off = (r16 * 16 + rr) * FB
                        for v in range(FB // 16):
                            a = accflat[pl.ds(off + v * 16, 16)]
                            accflat[pl.ds(off + v * 16, 16)] = a * dv
                    return 0
                lax.fori_loop(0, RANGE // 16, grow, 0)
                pltpu.sync_copy(
                    accflat.at[pl.ds(0, RANGE * FB)],
                    g_out.at[pl.ds(pl.multiple_of(
                        pass_ * N_P * FB + c0 * FB, 256), RANGE * FB)])

    return round_k


_round16 = _make_round(16, 1, True, False)
_round16_h = _make_round(16, 1, False, False)
_round128 = _make_round(32, 4, True, False)
_round128_h = _make_round(32, 4, False, True)


def _prep_body(deg_ref, x_ref, dis_ref, gx_ref):
    deg = deg_ref[...]
    dis = jnp.where(deg > 0, lax.rsqrt(deg), 0.0)
    dis_ref[...] = dis
    gx_ref[...] = x_ref[...] * dis[:, None]


def _prep(deg, x_p):
    return pl.pallas_call(
        _prep_body,
        grid=(N_P // BN,),
        in_specs=[pl.BlockSpec((BN,), lambda i: (i,)),
                  pl.BlockSpec((BN, 16), lambda i: (i, 0))],
        out_specs=[pl.BlockSpec((BN,), lambda i: (i,)),
                   pl.BlockSpec((BN, 16), lambda i: (i, 0))],
        out_shape=[jax.ShapeDtypeStruct((N_P,), jnp.float32),
                   jax.ShapeDtypeStruct((N_P, 16), jnp.float32)],
    )(deg, x_p)


def _dense1_body(x_ref, h1_ref, h2_ref, dis_ref, w0_ref, w1_ref, w2_ref,
                 b_ref, h_ref, gh_ref):
    acc = jnp.dot(x_ref[...], w0_ref[...], preferred_element_type=jnp.float32)
    acc += jnp.dot(h1_ref[...], w1_ref[...], preferred_element_type=jnp.float32)
    acc += jnp.dot(h2_ref[...], w2_ref[...], preferred_element_type=jnp.float32)
    acc += b_ref[...][None, :]
    h = jnp.maximum(acc, 0.0)
    h_ref[...] = h
    gh_ref[...] = h * dis_ref[...][:, None]


def _dense1(x_p, h1, h2, dis, w0, w1, w2, b):
    in16 = pl.BlockSpec((BN, 16), lambda i: (i, 0))
    w_spec = pl.BlockSpec((16, D_HID), lambda i: (0, 0))
    return pl.pallas_call(
        _dense1_body,
        grid=(N_P // BN,),
        in_specs=[in16, in16, in16, pl.BlockSpec((BN,), lambda i: (i,)),
                  w_spec, w_spec, w_spec,
                  pl.BlockSpec((D_HID,), lambda i: (0,))],
        out_specs=[pl.BlockSpec((BN, D_HID), lambda i: (i, 0)),
                   pl.BlockSpec((BN, D_HID), lambda i: (i, 0))],
        out_shape=[jax.ShapeDtypeStruct((N_P, D_HID), jnp.float32),
                   jax.ShapeDtypeStruct((N_P, D_HID), jnp.float32)],
    )(x_p, h1, h2, dis, w0, w1, w2, b)


def _dense2_body(h_ref, *refs):
    h21 = refs[0:4]
    h22 = refs[4:8]
    w0_ref = refs[8]
    w21 = refs[9:13]
    w22 = refs[13:17]
    b_ref = refs[17]
    o_ref = refs[18]
    acc = jnp.dot(h_ref[...], w0_ref[...], preferred_element_type=jnp.float32)
    for p in range(4):
        acc += jnp.dot(h21[p][0], w21[p][...],
                       preferred_element_type=jnp.float32)
        acc += jnp.dot(h22[p][0], w22[p][...],
                       preferred_element_type=jnp.float32)
    acc += b_ref[...][None, :]
    m = jnp.max(acc, axis=1, keepdims=True)
    lse = jnp.log(jnp.sum(jnp.exp(acc - m), axis=1, keepdims=True)) + m
    o_ref[...] = acc - lse


def _dense2(h, h21, h22, w0, w21s, w22s, b):
    in_h = pl.BlockSpec((BN, D_HID), lambda i: (i, 0))

    def pspec(p):
        return pl.BlockSpec((1, BN, 32), lambda i, p=p: (p, i, 0))

    wp_spec = pl.BlockSpec((32, D_OUT), lambda i: (0, 0))
    return pl.pallas_call(
        _dense2_body,
        grid=(N_P // BN,),
        in_specs=([in_h] + [pspec(p) for p in range(4)] * 2
                  + [pl.BlockSpec((D_HID, D_OUT), lambda i: (0, 0))]
                  + [wp_spec] * 8
                  + [pl.BlockSpec((D_OUT,), lambda i: (0,))]),
        out_specs=pl.BlockSpec((BN, D_OUT), lambda i: (i, 0)),
        out_shape=jax.ShapeDtypeStruct((N_P, D_OUT), jnp.float32),
    )(h, h21, h21, h21, h21, h22, h22, h22, h22,
      w0, *w21s, *w22s, b)


def kernel(x, edge_index, W1_0, W1_1, W1_2, b1, W2_0, W2_1, W2_2, b2):
    row = edge_index[0].astype(jnp.int32)
    col = edge_index[1].astype(jnp.int32)
    x_p = jnp.pad(x, ((0, N_P - N), (0, 16 - D_IN)))
    w1_0 = jnp.pad(W1_0, ((0, 16 - D_IN), (0, 0)))
    w1_1 = jnp.pad(W1_1, ((0, 16 - D_IN), (0, 0)))
    w1_2 = jnp.pad(W1_2, ((0, 16 - D_IN), (0, 0)))
    w21s = [W2_1[32 * p:32 * (p + 1)] for p in range(4)]
    w22s = [W2_2[32 * p:32 * (p + 1)] for p in range(4)]

    eown, deg = _phase_a(row, col)
    dis, gx = _prep(deg, x_p)

    h11_f, g11_f = _round16(gx, eown, dis)
    h11 = h11_f.reshape(N_P, 16)
    g11 = g11_f.reshape(N_P, 16)
    (h12_f,) = _round16_h(g11, eown, dis)
    h12 = h12_f.reshape(N_P, 16)
    h, gh = _dense1(x_p, h11, h12, dis, w1_0, w1_1, w1_2, b1)

    h21_f, g21_f = _round128(gh.reshape(N_P * 4, 32), eown, dis)
    h21 = h21_f.reshape(4, N_P, 32)
    (h22_f,) = _round128_h(g21_f.reshape(N_P * 4, 32), eown, dis)
    h22 = h22_f.reshape(4, N_P, 32)

    out = _dense2(h, h21, h22, W2_0, w21s, w22s, b2)
    return out[:N]


# speedup vs baseline: 9.5079x; 1.2798x over previous
import functools

import jax
import jax.numpy as jnp
from jax import lax
from jax.experimental import pallas as pl
from jax.experimental.pallas import tpu as pltpu
from jax.experimental.pallas import tpu_sc as plsc

N = 100000
E = 3200000
D_IN = 11
D_HID = 128
D_OUT = 2

NW = 32
RANGE = 3136
N_P = NW * RANGE
CAP = 102400
STG = 2048
CH = 4000
CH_D = 2048
ACC_ROWS = 3200
SENT_CL = RANGE
ECH = 128

BN = 1024

_mesh = plsc.VectorSubcoreMesh(
    core_axis_name="c", subcore_axis_name="s", num_cores=2, num_subcores=16)


def _wid():
    return lax.axis_index("s") * 2 + lax.axis_index("c")


PACK = 4096
INVALID = 0x7FFFFFFF


@functools.partial(
    pl.kernel,
    out_type=(
        jax.ShapeDtypeStruct((NW * CAP,), jnp.int32),
        jax.ShapeDtypeStruct((N_P,), jnp.float32),
    ),
    mesh=_mesh,
    compiler_params=pltpu.CompilerParams(
        needs_layout_passes=False, use_tc_tiling_on_sc=False),
    scratch_types=(
        pltpu.VMEM((2 * CH,), jnp.int32),
        pltpu.VMEM((2 * CH,), jnp.int32),
        pltpu.VMEM((2 * STG + 16,), jnp.int32),
        pltpu.VMEM((ACC_ROWS + 16,), jnp.float32),
        pltpu.SemaphoreType.DMA,
        pltpu.SemaphoreType.DMA,
    ),
)
def _phase_a(row_hbm, col_hbm, eown, deg_out,
             colbuf, rowbuf, stage, degbuf, asem0, asem1):
    wid = _wid()
    c0 = wid * RANGE
    iota = lax.broadcasted_iota(jnp.int32, (16,), 0)
    zero16 = jnp.zeros((16,), jnp.float32)
    one0 = jnp.where(iota == 0, 1.0, 0.0).astype(jnp.float32)
    asems = (asem0, asem1)

    def sent16(base):
        return ((iota * 521 + base + wid * 1031) & 32767) * PACK + SENT_CL

    def zdeg(j, _):
        degbuf[pl.ds(j * 16, 16)] = zero16
        return 0
    lax.fori_loop(0, (ACC_ROWS + 16) // 16, zdeg, 0)

    NCH = E // CH

    def load(i, par):
        off = par * CH
        pltpu.async_copy(col_hbm.at[pl.ds(pl.multiple_of(i * CH, 8), CH)],
                         colbuf.at[pl.ds(off, CH)], asems[par])
        pltpu.async_copy(row_hbm.at[pl.ds(pl.multiple_of(i * CH, 8), CH)],
                         rowbuf.at[pl.ds(off, CH)], asems[par])

    def drain(i, par):
        off = par * CH
        pltpu.make_async_copy(
            col_hbm.at[pl.ds(pl.multiple_of(i * CH, 8), CH)],
            colbuf.at[pl.ds(off, CH)], asems[par]).wait()
        pltpu.make_async_copy(
            row_hbm.at[pl.ds(pl.multiple_of(i * CH, 8), CH)],
            rowbuf.at[pl.ds(off, CH)], asems[par]).wait()

    load(0, 0)

    def chunk(i, carry):
        par_t = i & 1

        @pl.when((par_t == 0) & (i + 1 < NCH))
        def _():
            load(i + 1, 1)

        @pl.when((par_t == 1) & (i + 1 < NCH))
        def _():
            load(i + 1, 0)

        @pl.when(par_t == 0)
        def _():
            drain(i, 0)

        @pl.when(par_t == 1)
        def _():
            drain(i, 1)

        boff = par_t * CH

        def vec(j, pm):
            cv = colbuf[pl.ds(boff + j * 16, 16)]
            rv = rowbuf[pl.ds(boff + j * 16, 16)]
            lv = cv - c0
            m = plsc.bitcast(lv, jnp.uint32) < jnp.uint32(RANGE)
            packed = jnp.where(m, rv * PACK + lv, jnp.int32(INVALID))
            packed = jnp.sort(packed)
            stage[pl.ds(pm, 16)] = packed
            return pm + plsc.all_reduce_population_count(m)[0]

        pm, hp = carry
        pm = lax.fori_loop(0, CH // 16, vec, pm, unroll=4)
        do_flush = (pm >= STG) & (hp + STG <= CAP)

        @pl.when(do_flush)
        def _():
            pltpu.sync_copy(
                stage.at[pl.ds(0, STG)],
                eown.at[pl.ds(pl.multiple_of(wid * CAP + hp, STG), STG)])

            def mvv(t, _):
                stage[pl.ds(t * 16, 16)] = stage[pl.ds(STG + t * 16, 16)]
                return 0
            lax.fori_loop(0, (pm - STG + 15) // 16, mvv, 0)

        hp = jnp.where(do_flush, hp + STG, hp)
        pm = jnp.where(do_flush, pm - STG, pm)
        return (pm, hp)

    pm, hp = lax.fori_loop(0, NCH, chunk, (0, 0))

    do2 = (pm >= STG) & (hp + STG <= CAP)

    @pl.when(do2)
    def _():
        pltpu.sync_copy(
            stage.at[pl.ds(0, STG)],
            eown.at[pl.ds(pl.multiple_of(wid * CAP + hp, STG), STG)])

        def mvv2(t, _):
            stage[pl.ds(t * 16, 16)] = stage[pl.ds(STG + t * 16, 16)]
            return 0
        lax.fori_loop(0, (pm - STG + 15) // 16, mvv2, 0)

    hp = jnp.where(do2, hp + STG, hp)
    pm = jnp.where(do2, pm - STG, pm)

    def fin(j, _):
        base = j * 16
        keep = (iota + base) < pm
        cur = stage[pl.ds(base, 16)]
        stage[pl.ds(base, 16)] = jnp.where(keep, cur, sent16(base))
        return 0
    lax.fori_loop(0, STG // 16, fin, 0)

    def fill(i, hp2):
        @pl.when(hp2 < CAP)
        def _():
            pltpu.sync_copy(
                stage.at[pl.ds(0, STG)],
                eown.at[pl.ds(pl.multiple_of(wid * CAP + hp2, STG), STG)])

        @pl.when(i == 0)
        def _():
            def allsent(j, _):
                base = j * 16
                stage[pl.ds(base, 16)] = sent16(base)
                return 0
            lax.fori_loop(0, STG // 16, allsent, 0)

        return jnp.where(hp2 < CAP, hp2 + STG, hp2)
    lax.fori_loop(0, CAP // STG, fill, hp)

    def degchunk(i, _):
        pltpu.sync_copy(
            eown.at[pl.ds(pl.multiple_of(wid * CAP + i * CH_D, CH_D), CH_D)],
            colbuf.at[pl.ds(0, CH_D)])

        def grp(g, _):
            clv = colbuf[pl.ds(g * 16, 16)] & (PACK - 1)
            for ee in range(16):
                cl = clv[ee]
                w = degbuf[pl.ds(cl, 16)]
                degbuf[pl.ds(cl, 16)] = w + one0
            return 0
        lax.fori_loop(0, CH_D // 16, grp, 0)
        return 0
    lax.fori_loop(0, CAP // CH_D, degchunk, 0)

    pltpu.sync_copy(degbuf.at[pl.ds(0, RANGE)],
                    deg_out.at[pl.ds(pl.multiple_of(wid * RANGE, 64), RANGE)])


K_PIPE = 2
GRP = ECH * K_PIPE
NG = CAP // GRP
ACC_R = 3152


def _make_round(FB, PP, want_g, blocked_src):
    out_type = [jax.ShapeDtypeStruct((PP * N_P * FB,), jnp.float32)]
    if want_g:
        out_type.append(jax.ShapeDtypeStruct((PP * N_P * FB,), jnp.float32))

    @functools.partial(
        pl.kernel,
        out_type=tuple(out_type),
        mesh=_mesh,
        compiler_params=pltpu.CompilerParams(
            needs_layout_passes=False, use_tc_tiling_on_sc=False),
        scratch_types=(
            pltpu.VMEM((2 * GRP,), jnp.int32),
            pltpu.VMEM((2 * GRP,), jnp.int32),
            pltpu.VMEM((2 * GRP, FB), jnp.float32),
            pltpu.VMEM((ACC_R * FB,), jnp.float32),
            pltpu.VMEM((RANGE,), jnp.float32),
            pltpu.SemaphoreType.DMA,
            pltpu.SemaphoreType.DMA,
        ),
    )
    def round_k(g_hbm, eown, dis_hbm, *rest):
        if want_g:
            h_out, g_out = rest[0], rest[1]
            scratch = rest[2:]
        else:
            h_out = rest[0]
            scratch = rest[1:]
        ebuf, idxbuf, msgbuf, accflat, disbuf, gsem0, gsem1 = scratch
        gsems = (gsem0, gsem1)
        wid = _wid()
        c0 = wid * RANGE
        inf16 = jnp.full((16,), jnp.inf, jnp.float32)

        pltpu.sync_copy(dis_hbm.at[pl.ds(pl.multiple_of(c0, 64), RANGE)],
                        disbuf)

        for pass_ in range(PP):
            def fire(gi, par):
                off = par * GRP
                pltpu.sync_copy(
                    eown.at[pl.ds(pl.multiple_of(wid * CAP + gi * GRP, GRP),
                                  GRP)],
                    ebuf.at[pl.ds(off, GRP)])

                def mkidx(v, _):
                    rv = lax.shift_right_logical(
                        ebuf[pl.ds(off + v * 16, 16)], 12)
                    if blocked_src:
                        idxbuf[pl.ds(off + v * 16, 16)] = rv + pass_ * N_P
                    else:
                        idxbuf[pl.ds(off + v * 16, 16)] = rv * PP + pass_
                    return 0
                lax.fori_loop(0, GRP // 16, mkidx, 0)
                for k in range(K_PIPE):
                    pltpu.async_copy(
                        g_hbm.at[idxbuf.at[pl.ds(off + k * ECH, ECH)]],
                        msgbuf.at[pl.ds(off + k * ECH, ECH)],
                        gsems[par])

            def drain_process(par):
                off = par * GRP
                for k in range(K_PIPE):
                    pltpu.make_async_copy(
                        g_hbm.at[idxbuf.at[pl.ds(off + k * ECH, ECH)]],
                        msgbuf.at[pl.ds(off + k * ECH, ECH)],
                        gsems[par]).wait()

                def grp(g, _):
                    clv = ebuf[pl.ds(off + g * 16, 16)] & (PACK - 1)
                    for ee in range(16):
                        base = clv[ee] * FB
                        r = off + g * 16 + ee
                        for v in range(FB // 16):
                            a = accflat[pl.ds(base + v * 16, 16)]
                            mv = msgbuf[r, pl.ds(v * 16, 16)]
                            accflat[pl.ds(base + v * 16, 16)] = \
                                jnp.minimum(a, mv)
                    return 0
                lax.fori_loop(0, GRP // 16, grp, 0)

            def zacc(j, _):
                accflat[pl.ds(j * 16, 16)] = inf16
                return 0
            lax.fori_loop(0, ACC_R * FB // 16, zacc, 0)

            fire(0, 0)

            def body(g, _):
                par_t = g & 1

                @pl.when((par_t == 0) & (g + 1 < NG))
                def _():
                    fire(g + 1, 1)

                @pl.when((par_t == 1) & (g + 1 < NG))
                def _():
                    fire(g + 1, 0)

                @pl.when(par_t == 0)
                def _():
                    drain_process(0)

                @pl.when(par_t == 1)
                def _():
                    drain_process(1)
                return 0
            lax.fori_loop(0, NG, body, 0)

            def hrow(r16, _):
                dvec = disbuf[pl.ds(r16 * 16, 16)]
                for rr in range(16):
                    dv = dvec[rr]
                    off = (r16 * 16 + rr) * FB
                    for v in range(FB // 16):
                        a = accflat[pl.ds(off + v * 16, 16)]
                        a = jnp.where(a == jnp.inf, 0.0, a) * dv
                        accflat[pl.ds(off + v * 16, 16)] = a
                return 0
            lax.fori_loop(0, RANGE // 16, hrow, 0)
            pltpu.sync_copy(
                accflat.at[pl.ds(0, RANGE * FB)],
                h_out.at[pl.ds(pl.multiple_of(
                    pass_ * N_P * FB + c0 * FB, 256), RANGE * FB)])

            if want_g:
                def grow(r16, _):
                    dvec = disbuf[pl.ds(r16 * 16, 16)]
                    for rr in range(16):
                        dv = dvec[rr]
                        off = (r16 * 16 + rr) * FB
                        for v in range(FB // 16):
                            a = accflat[pl.ds(off + v * 16, 16)]
                            accflat[pl.ds(off + v * 16, 16)] = a * dv
                    return 0
                lax.fori_loop(0, RANGE // 16, grow, 0)
                pltpu.sync_copy(
                    accflat.at[pl.ds(0, RANGE * FB)],
                    g_out.at[pl.ds(pl.multiple_of(
                        pass_ * N_P * FB + c0 * FB, 256), RANGE * FB)])

    return round_k


_round16 = _make_round(16, 1, True, False)
_round16_h = _make_round(16, 1, False, False)
_round128 = _make_round(32, 4, True, False)
_round128_h = _make_round(32, 4, False, True)


def _prep_body(deg_ref, x_ref, dis_ref, gx_ref):
    deg = deg_ref[...]
    dis = jnp.where(deg > 0, lax.rsqrt(deg), 0.0)
    dis_ref[...] = dis
    gx_ref[...] = x_ref[...] * dis[:, None]


def _prep(deg, x_p):
    return pl.pallas_call(
        _prep_body,
        grid=(N_P // BN,),
        in_specs=[pl.BlockSpec((BN,), lambda i: (i,)),
                  pl.BlockSpec((BN, 16), lambda i: (i, 0))],
        out_specs=[pl.BlockSpec((BN,), lambda i: (i,)),
                   pl.BlockSpec((BN, 16), lambda i: (i, 0))],
        out_shape=[jax.ShapeDtypeStruct((N_P,), jnp.float32),
                   jax.ShapeDtypeStruct((N_P, 16), jnp.float32)],
    )(deg, x_p)


def _dense1_body(x_ref, h1_ref, h2_ref, dis_ref, w0_ref, w1_ref, w2_ref,
                 b_ref, h_ref, gh_ref):
    acc = jnp.dot(x_ref[...], w0_ref[...], preferred_element_type=jnp.float32)
    acc += jnp.dot(h1_ref[...], w1_ref[...], preferred_element_type=jnp.float32)
    acc += jnp.dot(h2_ref[...], w2_ref[...], preferred_element_type=jnp.float32)
    acc += b_ref[...][None, :]
    h = jnp.maximum(acc, 0.0)
    h_ref[...] = h
    gh_ref[...] = h * dis_ref[...][:, None]


def _dense1(x_p, h1, h2, dis, w0, w1, w2, b):
    in16 = pl.BlockSpec((BN, 16), lambda i: (i, 0))
    w_spec = pl.BlockSpec((16, D_HID), lambda i: (0, 0))
    return pl.pallas_call(
        _dense1_body,
        grid=(N_P // BN,),
        in_specs=[in16, in16, in16, pl.BlockSpec((BN,), lambda i: (i,)),
                  w_spec, w_spec, w_spec,
                  pl.BlockSpec((D_HID,), lambda i: (0,))],
        out_specs=[pl.BlockSpec((BN, D_HID), lambda i: (i, 0)),
                   pl.BlockSpec((BN, D_HID), lambda i: (i, 0))],
        out_shape=[jax.ShapeDtypeStruct((N_P, D_HID), jnp.float32),
                   jax.ShapeDtypeStruct((N_P, D_HID), jnp.float32)],
    )(x_p, h1, h2, dis, w0, w1, w2, b)


def _dense2_body(h_ref, *refs):
    h21 = refs[0:4]
    h22 = refs[4:8]
    w0_ref = refs[8]
    w21 = refs[9:13]
    w22 = refs[13:17]
    b_ref = refs[17]
    o_ref = refs[18]
    acc = jnp.dot(h_ref[...], w0_ref[...], preferred_element_type=jnp.float32)
    for p in range(4):
        acc += jnp.dot(h21[p][0], w21[p][...],
                       preferred_element_type=jnp.float32)
        acc += jnp.dot(h22[p][0], w22[p][...],
                       preferred_element_type=jnp.float32)
    acc += b_ref[...][None, :]
    m = jnp.max(acc, axis=1, keepdims=True)
    lse = jnp.log(jnp.sum(jnp.exp(acc - m), axis=1, keepdims=True)) + m
    o_ref[...] = acc - lse


def _dense2(h, h21, h22, w0, w21s, w22s, b):
    in_h = pl.BlockSpec((BN, D_HID), lambda i: (i, 0))

    def pspec(p):
        return pl.BlockSpec((1, BN, 32), lambda i, p=p: (p, i, 0))

    wp_spec = pl.BlockSpec((32, D_OUT), lambda i: (0, 0))
    return pl.pallas_call(
        _dense2_body,
        grid=(N_P // BN,),
        in_specs=([in_h] + [pspec(p) for p in range(4)] * 2
                  + [pl.BlockSpec((D_HID, D_OUT), lambda i: (0, 0))]
                  + [wp_spec] * 8
                  + [pl.BlockSpec((D_OUT,), lambda i: (0,))]),
        out_specs=pl.BlockSpec((BN, D_OUT), lambda i: (i, 0)),
        out_shape=jax.ShapeDtypeStruct((N_P, D_OUT), jnp.float32),
    )(h, h21, h21, h21, h21, h22, h22, h22, h22,
      w0, *w21s, *w22s, b)


def kernel(x, edge_index, W1_0, W1_1, W1_2, b1, W2_0, W2_1, W2_2, b2):
    row = edge_index[0].astype(jnp.int32)
    col = edge_index[1].astype(jnp.int32)
    x_p = jnp.pad(x, ((0, N_P - N), (0, 16 - D_IN)))
    w1_0 = jnp.pad(W1_0, ((0, 16 - D_IN), (0, 0)))
    w1_1 = jnp.pad(W1_1, ((0, 16 - D_IN), (0, 0)))
    w1_2 = jnp.pad(W1_2, ((0, 16 - D_IN), (0, 0)))
    w21s = [W2_1[32 * p:32 * (p + 1)] for p in range(4)]
    w22s = [W2_2[32 * p:32 * (p + 1)] for p in range(4)]

    eown, deg = _phase_a(row, col)
    dis, gx = _prep(deg, x_p)

    h11_f, g11_f = _round16(gx, eown, dis)
    h11 = h11_f.reshape(N_P, 16)
    g11 = g11_f.reshape(N_P, 16)
    (h12_f,) = _round16_h(g11, eown, dis)
    h12 = h12_f.reshape(N_P, 16)
    h, gh = _dense1(x_p, h11, h12, dis, w1_0, w1_1, w1_2, b1)

    h21_f, g21_f = _round128(gh.reshape(N_P * 4, 32), eown, dis)
    h21 = h21_f.reshape(4, N_P, 32)
    (h22_f,) = _round128_h(g21_f.reshape(N_P * 4, 32), eown, dis)
    h22 = h22_f.reshape(4, N_P, 32)

    out = _dense2(h, h21, h22, W2_0, w21s, w22s, b2)
    return out[:N]


# speedup vs baseline: 10.7828x; 1.1341x over previous
import functools

import jax
import jax.numpy as jnp
from jax import lax
from jax.experimental import pallas as pl
from jax.experimental.pallas import tpu as pltpu
from jax.experimental.pallas import tpu_sc as plsc

N = 100000
E = 3200000
D_IN = 11
D_HID = 128
D_OUT = 2

NW = 32
RANGE = 3136
N_P = NW * RANGE
CAP = 102400
STG = 2048
CH = 4000
CH_D = 2048
ACC_ROWS = 3200
SENT_CL = RANGE
ECH = 128

BN = 1024

_mesh = plsc.VectorSubcoreMesh(
    core_axis_name="c", subcore_axis_name="s", num_cores=2, num_subcores=16)


def _wid():
    return lax.axis_index("s") * 2 + lax.axis_index("c")


PACK = 4096
INVALID = 0x7FFFFFFF


@functools.partial(
    pl.kernel,
    out_type=(
        jax.ShapeDtypeStruct((NW * CAP,), jnp.int32),
        jax.ShapeDtypeStruct((N_P,), jnp.float32),
    ),
    mesh=_mesh,
    compiler_params=pltpu.CompilerParams(
        needs_layout_passes=False, use_tc_tiling_on_sc=False),
    scratch_types=(
        pltpu.VMEM((2 * CH,), jnp.int32),
        pltpu.VMEM((2 * CH,), jnp.int32),
        pltpu.VMEM((2 * STG + 16,), jnp.int32),
        pltpu.VMEM((ACC_ROWS + 16,), jnp.float32),
        pltpu.SemaphoreType.DMA,
        pltpu.SemaphoreType.DMA,
    ),
)
def _phase_a(row_hbm, col_hbm, eown, deg_out,
             colbuf, rowbuf, stage, degbuf, asem0, asem1):
    wid = _wid()
    c0 = wid * RANGE
    iota = lax.broadcasted_iota(jnp.int32, (16,), 0)
    zero16 = jnp.zeros((16,), jnp.float32)
    one0 = jnp.where(iota == 0, 1.0, 0.0).astype(jnp.float32)
    asems = (asem0, asem1)

    def sent16(base):
        return ((iota * 521 + base + wid * 1031) & 32767) * PACK + SENT_CL

    def zdeg(j, _):
        degbuf[pl.ds(j * 16, 16)] = zero16
        return 0
    lax.fori_loop(0, (ACC_ROWS + 16) // 16, zdeg, 0)

    NCH = E // CH

    def load(i, par):
        off = par * CH
        pltpu.async_copy(col_hbm.at[pl.ds(pl.multiple_of(i * CH, 8), CH)],
                         colbuf.at[pl.ds(off, CH)], asems[par])
        pltpu.async_copy(row_hbm.at[pl.ds(pl.multiple_of(i * CH, 8), CH)],
                         rowbuf.at[pl.ds(off, CH)], asems[par])

    def drain(i, par):
        off = par * CH
        pltpu.make_async_copy(
            col_hbm.at[pl.ds(pl.multiple_of(i * CH, 8), CH)],
            colbuf.at[pl.ds(off, CH)], asems[par]).wait()
        pltpu.make_async_copy(
            row_hbm.at[pl.ds(pl.multiple_of(i * CH, 8), CH)],
            rowbuf.at[pl.ds(off, CH)], asems[par]).wait()

    load(0, 0)

    def chunk(i, carry):
        par_t = i & 1

        @pl.when((par_t == 0) & (i + 1 < NCH))
        def _():
            load(i + 1, 1)

        @pl.when((par_t == 1) & (i + 1 < NCH))
        def _():
            load(i + 1, 0)

        @pl.when(par_t == 0)
        def _():
            drain(i, 0)

        @pl.when(par_t == 1)
        def _():
            drain(i, 1)

        boff = par_t * CH

        def vec(j, pm):
            cv = colbuf[pl.ds(boff + j * 16, 16)]
            rv = rowbuf[pl.ds(boff + j * 16, 16)]
            lv = cv - c0
            m = plsc.bitcast(lv, jnp.uint32) < jnp.uint32(RANGE)
            packed = jnp.where(m, rv * PACK + lv, jnp.int32(INVALID))
            packed = jnp.sort(packed)
            stage[pl.ds(pm, 16)] = packed
            return pm + plsc.all_reduce_population_count(m)[0]

        pm, hp = carry
        pm = lax.fori_loop(0, CH // 16, vec, pm, unroll=4)
        do_flush = (pm >= STG) & (hp + STG <= CAP)

        @pl.when(do_flush)
        def _():
            pltpu.sync_copy(
                stage.at[pl.ds(0, STG)],
                eown.at[pl.ds(pl.multiple_of(wid * CAP + hp, STG), STG)])

            def mvv(t, _):
                stage[pl.ds(t * 16, 16)] = stage[pl.ds(STG + t * 16, 16)]
                return 0
            lax.fori_loop(0, (pm - STG + 15) // 16, mvv, 0)

        hp = jnp.where(do_flush, hp + STG, hp)
        pm = jnp.where(do_flush, pm - STG, pm)
        return (pm, hp)

    pm, hp = lax.fori_loop(0, NCH, chunk, (0, 0))

    do2 = (pm >= STG) & (hp + STG <= CAP)

    @pl.when(do2)
    def _():
        pltpu.sync_copy(
            stage.at[pl.ds(0, STG)],
            eown.at[pl.ds(pl.multiple_of(wid * CAP + hp, STG), STG)])

        def mvv2(t, _):
            stage[pl.ds(t * 16, 16)] = stage[pl.ds(STG + t * 16, 16)]
            return 0
        lax.fori_loop(0, (pm - STG + 15) // 16, mvv2, 0)

    hp = jnp.where(do2, hp + STG, hp)
    pm = jnp.where(do2, pm - STG, pm)

    def fin(j, _):
        base = j * 16
        keep = (iota + base) < pm
        cur = stage[pl.ds(base, 16)]
        stage[pl.ds(base, 16)] = jnp.where(keep, cur, sent16(base))
        return 0
    lax.fori_loop(0, STG // 16, fin, 0)

    def fill(i, hp2):
        @pl.when(hp2 < CAP)
        def _():
            pltpu.sync_copy(
                stage.at[pl.ds(0, STG)],
                eown.at[pl.ds(pl.multiple_of(wid * CAP + hp2, STG), STG)])

        @pl.when(i == 0)
        def _():
            def allsent(j, _):
                base = j * 16
                stage[pl.ds(base, 16)] = sent16(base)
                return 0
            lax.fori_loop(0, STG // 16, allsent, 0)

        return jnp.where(hp2 < CAP, hp2 + STG, hp2)
    lax.fori_loop(0, CAP // STG, fill, hp)

    def degchunk(i, _):
        pltpu.sync_copy(
            eown.at[pl.ds(pl.multiple_of(wid * CAP + i * CH_D, CH_D), CH_D)],
            colbuf.at[pl.ds(0, CH_D)])

        def grp(g, _):
            clv = colbuf[pl.ds(g * 16, 16)] & (PACK - 1)
            for ee in range(16):
                cl = clv[ee]
                w = degbuf[pl.ds(cl, 16)]
                degbuf[pl.ds(cl, 16)] = w + one0
            return 0
        lax.fori_loop(0, CH_D // 16, grp, 0)
        return 0
    lax.fori_loop(0, CAP // CH_D, degchunk, 0)

    pltpu.sync_copy(degbuf.at[pl.ds(0, RANGE)],
                    deg_out.at[pl.ds(pl.multiple_of(wid * RANGE, 64), RANGE)])


K_PIPE = 2
GRP = ECH * K_PIPE
NG = CAP // GRP
ACC_R = 3152


def _make_round(FB, PP, want_g, blocked_src):
    out_type = [jax.ShapeDtypeStruct((PP * N_P * FB,), jnp.float32)]
    if want_g:
        out_type.append(jax.ShapeDtypeStruct((PP * N_P * FB,), jnp.float32))

    @functools.partial(
        pl.kernel,
        out_type=tuple(out_type),
        mesh=_mesh,
        compiler_params=pltpu.CompilerParams(
            needs_layout_passes=False, use_tc_tiling_on_sc=False),
        scratch_types=(
            pltpu.VMEM((2 * GRP,), jnp.int32),
            pltpu.VMEM((2 * GRP,), jnp.int32),
            pltpu.VMEM((2 * GRP,), jnp.int32),
            pltpu.VMEM((2 * GRP, FB), jnp.float32),
            pltpu.VMEM((ACC_R * FB,), jnp.float32),
            pltpu.VMEM((RANGE,), jnp.float32),
            pltpu.SemaphoreType.DMA,
            pltpu.SemaphoreType.DMA,
            pltpu.SemaphoreType.DMA,
            pltpu.SemaphoreType.DMA,
        ),
    )
    def round_k(g_hbm, eown, dis_hbm, *rest):
        if want_g:
            h_out, g_out = rest[0], rest[1]
            scratch = rest[2:]
        else:
            h_out = rest[0]
            scratch = rest[1:]
        (ebuf, idxbuf, clbuf, msgbuf, accflat, disbuf,
         gsem0, gsem1, esem0, esem1) = scratch
        gsems = (gsem0, gsem1)
        esems = (esem0, esem1)
        wid = _wid()
        c0 = wid * RANGE
        inf16 = jnp.full((16,), jnp.inf, jnp.float32)

        pltpu.sync_copy(dis_hbm.at[pl.ds(pl.multiple_of(c0, 64), RANGE)],
                        disbuf)

        for pass_ in range(PP):
            def eload(gi, par):
                off = par * GRP
                pltpu.async_copy(
                    eown.at[pl.ds(pl.multiple_of(wid * CAP + gi * GRP, GRP),
                                  GRP)],
                    ebuf.at[pl.ds(off, GRP)], esems[par])

            def launch(gi, par):
                off = par * GRP
                pltpu.make_async_copy(
                    eown.at[pl.ds(pl.multiple_of(wid * CAP + gi * GRP, GRP),
                                  GRP)],
                    ebuf.at[pl.ds(off, GRP)], esems[par]).wait()

                def mkidx(v, _):
                    ev = ebuf[pl.ds(off + v * 16, 16)]
                    rv = lax.shift_right_logical(ev, 12)
                    clbuf[pl.ds(off + v * 16, 16)] = ev & (PACK - 1)
                    if blocked_src:
                        idxbuf[pl.ds(off + v * 16, 16)] = rv + pass_ * N_P
                    else:
                        idxbuf[pl.ds(off + v * 16, 16)] = rv * PP + pass_
                    return 0
                lax.fori_loop(0, GRP // 16, mkidx, 0)
                for k in range(K_PIPE):
                    pltpu.async_copy(
                        g_hbm.at[idxbuf.at[pl.ds(off + k * ECH, ECH)]],
                        msgbuf.at[pl.ds(off + k * ECH, ECH)],
                        gsems[par])

            def drain_process(par):
                off = par * GRP
                for k in range(K_PIPE):
                    pltpu.make_async_copy(
                        g_hbm.at[idxbuf.at[pl.ds(off + k * ECH, ECH)]],
                        msgbuf.at[pl.ds(off + k * ECH, ECH)],
                        gsems[par]).wait()

                def grp(g, _):
                    clv = clbuf[pl.ds(off + g * 16, 16)]
                    for ee in range(16):
                        base = clv[ee] * FB
                        r = off + g * 16 + ee
                        for v in range(FB // 16):
                            a = accflat[pl.ds(base + v * 16, 16)]
                            mv = msgbuf[r, pl.ds(v * 16, 16)]
                            accflat[pl.ds(base + v * 16, 16)] = \
                                jnp.minimum(a, mv)
                    return 0
                lax.fori_loop(0, GRP // 16, grp, 0)

            def zacc(j, _):
                accflat[pl.ds(j * 16, 16)] = inf16
                return 0
            lax.fori_loop(0, ACC_R * FB // 16, zacc, 0)

            eload(0, 0)
            launch(0, 0)
            eload(1, 1)

            def body(g, _):
                par_t = g & 1

                @pl.when((par_t == 0) & (g + 1 < NG))
                def _():
                    launch(g + 1, 1)

                @pl.when((par_t == 1) & (g + 1 < NG))
                def _():
                    launch(g + 1, 0)

                @pl.when((par_t == 0) & (g + 2 < NG))
                def _():
                    eload(g + 2, 0)

                @pl.when((par_t == 1) & (g + 2 < NG))
                def _():
                    eload(g + 2, 1)

                @pl.when(par_t == 0)
                def _():
                    drain_process(0)

                @pl.when(par_t == 1)
                def _():
                    drain_process(1)
                return 0
            lax.fori_loop(0, NG, body, 0)

            def hrow(r16, _):
                dvec = disbuf[pl.ds(r16 * 16, 16)]
                for rr in range(16):
                    dv = dvec[rr]
                    off = (r16 * 16 + rr) * FB
                    for v in range(FB // 16):
                        a = accflat[pl.ds(off + v * 16, 16)]
                        a = jnp.where(a == jnp.inf, 0.0, a) * dv
                        accflat[pl.ds(off + v * 16, 16)] = a
                return 0
            lax.fori_loop(0, RANGE // 16, hrow, 0)
            pltpu.sync_copy(
                accflat.at[pl.ds(0, RANGE * FB)],
                h_out.at[pl.ds(pl.multiple_of(
                    pass_ * N_P * FB + c0 * FB, 256), RANGE * FB)])

            if want_g:
                def grow(r16, _):
                    dvec = disbuf[pl.ds(r16 * 16, 16)]
                    for rr in range(16):
                        dv = dvec[rr]
                        off = (r16 * 16 + rr) * FB
                        for v in range(FB // 16):
                            a = accflat[pl.ds(off + v * 16, 16)]
                            accflat[pl.ds(off + v * 16, 16)] = a * dv
                    return 0
                lax.fori_loop(0, RANGE // 16, grow, 0)
                pltpu.sync_copy(
                    accflat.at[pl.ds(0, RANGE * FB)],
                    g_out.at[pl.ds(pl.multiple_of(
                        pass_ * N_P * FB + c0 * FB, 256), RANGE * FB)])

    return round_k


_round16 = _make_round(16, 1, True, False)
_round16_h = _make_round(16, 1, False, False)
_round128 = _make_round(32, 4, True, False)
_round128_h = _make_round(32, 4, False, True)


def _prep_body(deg_ref, x_ref, dis_ref, gx_ref):
    deg = deg_ref[...]
    dis = jnp.where(deg > 0, lax.rsqrt(deg), 0.0)
    dis_ref[...] = dis
    gx_ref[...] = x_ref[...] * dis[:, None]


def _prep(deg, x_p):
    return pl.pallas_call(
        _prep_body,
        grid=(N_P // BN,),
        in_specs=[pl.BlockSpec((BN,), lambda i: (i,)),
                  pl.BlockSpec((BN, 16), lambda i: (i, 0))],
        out_specs=[pl.BlockSpec((BN,), lambda i: (i,)),
                   pl.BlockSpec((BN, 16), lambda i: (i, 0))],
        out_shape=[jax.ShapeDtypeStruct((N_P,), jnp.float32),
                   jax.ShapeDtypeStruct((N_P, 16), jnp.float32)],
    )(deg, x_p)


def _dense1_body(x_ref, h1_ref, h2_ref, dis_ref, w0_ref, w1_ref, w2_ref,
                 b_ref, h_ref, gh_ref):
    acc = jnp.dot(x_ref[...], w0_ref[...], preferred_element_type=jnp.float32)
    acc += jnp.dot(h1_ref[...], w1_ref[...], preferred_element_type=jnp.float32)
    acc += jnp.dot(h2_ref[...], w2_ref[...], preferred_element_type=jnp.float32)
    acc += b_ref[...][None, :]
    h = jnp.maximum(acc, 0.0)
    h_ref[...] = h
    gh_ref[...] = h * dis_ref[...][:, None]


def _dense1(x_p, h1, h2, dis, w0, w1, w2, b):
    in16 = pl.BlockSpec((BN, 16), lambda i: (i, 0))
    w_spec = pl.BlockSpec((16, D_HID), lambda i: (0, 0))
    return pl.pallas_call(
        _dense1_body,
        grid=(N_P // BN,),
        in_specs=[in16, in16, in16, pl.BlockSpec((BN,), lambda i: (i,)),
                  w_spec, w_spec, w_spec,
                  pl.BlockSpec((D_HID,), lambda i: (0,))],
        out_specs=[pl.BlockSpec((BN, D_HID), lambda i: (i, 0)),
                   pl.BlockSpec((BN, D_HID), lambda i: (i, 0))],
        out_shape=[jax.ShapeDtypeStruct((N_P, D_HID), jnp.float32),
                   jax.ShapeDtypeStruct((N_P, D_HID), jnp.float32)],
    )(x_p, h1, h2, dis, w0, w1, w2, b)


def _dense2_body(h_ref, *refs):
    h21 = refs[0:4]
    h22 = refs[4:8]
    w0_ref = refs[8]
    w21 = refs[9:13]
    w22 = refs[13:17]
    b_ref = refs[17]
    o_ref = refs[18]
    acc = jnp.dot(h_ref[...], w0_ref[...], preferred_element_type=jnp.float32)
    for p in range(4):
        acc += jnp.dot(h21[p][0], w21[p][...],
                       preferred_element_type=jnp.float32)
        acc += jnp.dot(h22[p][0], w22[p][...],
                       preferred_element_type=jnp.float32)
    acc += b_ref[...][None, :]
    m = jnp.max(acc, axis=1, keepdims=True)
    lse = jnp.log(jnp.sum(jnp.exp(acc - m), axis=1, keepdims=True)) + m
    o_ref[...] = acc - lse


def _dense2(h, h21, h22, w0, w21s, w22s, b):
    in_h = pl.BlockSpec((BN, D_HID), lambda i: (i, 0))

    def pspec(p):
        return pl.BlockSpec((1, BN, 32), lambda i, p=p: (p, i, 0))

    wp_spec = pl.BlockSpec((32, D_OUT), lambda i: (0, 0))
    return pl.pallas_call(
        _dense2_body,
        grid=(N_P // BN,),
        in_specs=([in_h] + [pspec(p) for p in range(4)] * 2
                  + [pl.BlockSpec((D_HID, D_OUT), lambda i: (0, 0))]
                  + [wp_spec] * 8
                  + [pl.BlockSpec((D_OUT,), lambda i: (0,))]),
        out_specs=pl.BlockSpec((BN, D_OUT), lambda i: (i, 0)),
        out_shape=jax.ShapeDtypeStruct((N_P, D_OUT), jnp.float32),
    )(h, h21, h21, h21, h21, h22, h22, h22, h22,
      w0, *w21s, *w22s, b)


def kernel(x, edge_index, W1_0, W1_1, W1_2, b1, W2_0, W2_1, W2_2, b2):
    row = edge_index[0].astype(jnp.int32)
    col = edge_index[1].astype(jnp.int32)
    x_p = jnp.pad(x, ((0, N_P - N), (0, 16 - D_IN)))
    w1_0 = jnp.pad(W1_0, ((0, 16 - D_IN), (0, 0)))
    w1_1 = jnp.pad(W1_1, ((0, 16 - D_IN), (0, 0)))
    w1_2 = jnp.pad(W1_2, ((0, 16 - D_IN), (0, 0)))
    w21s = [W2_1[32 * p:32 * (p + 1)] for p in range(4)]
    w22s = [W2_2[32 * p:32 * (p + 1)] for p in range(4)]

    eown, deg = _phase_a(row, col)
    dis, gx = _prep(deg, x_p)

    h11_f, g11_f = _round16(gx, eown, dis)
    h11 = h11_f.reshape(N_P, 16)
    g11 = g11_f.reshape(N_P, 16)
    (h12_f,) = _round16_h(g11, eown, dis)
    h12 = h12_f.reshape(N_P, 16)
    h, gh = _dense1(x_p, h11, h12, dis, w1_0, w1_1, w1_2, b1)

    h21_f, g21_f = _round128(gh.reshape(N_P * 4, 32), eown, dis)
    h21 = h21_f.reshape(4, N_P, 32)
    (h22_f,) = _round128_h(g21_f.reshape(N_P * 4, 32), eown, dis)
    h22 = h22_f.reshape(4, N_P, 32)

    out = _dense2(h, h21, h22, W2_0, w21s, w22s, b2)
    return out[:N]


# speedup vs baseline: 10.8609x; 1.0072x over previous
import functools

import jax
import jax.numpy as jnp
from jax import lax
from jax.experimental import pallas as pl
from jax.experimental.pallas import tpu as pltpu
from jax.experimental.pallas import tpu_sc as plsc

N = 100000
E = 3200000
D_IN = 11
D_HID = 128
D_OUT = 2

NW = 32
RANGE = 3136
N_P = NW * RANGE
CAP = 102400
STG = 2048
CH = 4000
CH_D = 2048
ACC_ROWS = 3200
SENT_CL = RANGE
ECH = 128

BN = 1024

_mesh = plsc.VectorSubcoreMesh(
    core_axis_name="c", subcore_axis_name="s", num_cores=2, num_subcores=16)


def _wid():
    return lax.axis_index("s") * 2 + lax.axis_index("c")


PACK = 4096
INVALID = 0x7FFFFFFF


@functools.partial(
    pl.kernel,
    out_type=(
        jax.ShapeDtypeStruct((NW * CAP,), jnp.int32),
        jax.ShapeDtypeStruct((N_P,), jnp.float32),
    ),
    mesh=_mesh,
    compiler_params=pltpu.CompilerParams(
        needs_layout_passes=False, use_tc_tiling_on_sc=False),
    scratch_types=(
        pltpu.VMEM((2 * CH,), jnp.int32),
        pltpu.VMEM((2 * CH,), jnp.int32),
        pltpu.VMEM((2 * STG + 16,), jnp.int32),
        pltpu.VMEM((ACC_ROWS + 16,), jnp.float32),
        pltpu.SemaphoreType.DMA,
        pltpu.SemaphoreType.DMA,
    ),
)
def _phase_a(row_hbm, col_hbm, eown, deg_out,
             colbuf, rowbuf, stage, degbuf, asem0, asem1):
    wid = _wid()
    c0 = wid * RANGE
    iota = lax.broadcasted_iota(jnp.int32, (16,), 0)
    zero16 = jnp.zeros((16,), jnp.float32)
    one0 = jnp.where(iota == 0, 1.0, 0.0).astype(jnp.float32)
    asems = (asem0, asem1)

    def sent16(base):
        return ((iota * 521 + base + wid * 1031) & 32767) * PACK + SENT_CL

    def zdeg(j, _):
        degbuf[pl.ds(j * 16, 16)] = zero16
        return 0
    lax.fori_loop(0, (ACC_ROWS + 16) // 16, zdeg, 0)

    NCH = E // CH

    def load(i, par):
        off = par * CH
        pltpu.async_copy(col_hbm.at[pl.ds(pl.multiple_of(i * CH, 8), CH)],
                         colbuf.at[pl.ds(off, CH)], asems[par])
        pltpu.async_copy(row_hbm.at[pl.ds(pl.multiple_of(i * CH, 8), CH)],
                         rowbuf.at[pl.ds(off, CH)], asems[par])

    def drain(i, par):
        off = par * CH
        pltpu.make_async_copy(
            col_hbm.at[pl.ds(pl.multiple_of(i * CH, 8), CH)],
            colbuf.at[pl.ds(off, CH)], asems[par]).wait()
        pltpu.make_async_copy(
            row_hbm.at[pl.ds(pl.multiple_of(i * CH, 8), CH)],
            rowbuf.at[pl.ds(off, CH)], asems[par]).wait()

    load(0, 0)

    def chunk(i, carry):
        par_t = i & 1

        @pl.when((par_t == 0) & (i + 1 < NCH))
        def _():
            load(i + 1, 1)

        @pl.when((par_t == 1) & (i + 1 < NCH))
        def _():
            load(i + 1, 0)

        @pl.when(par_t == 0)
        def _():
            drain(i, 0)

        @pl.when(par_t == 1)
        def _():
            drain(i, 1)

        boff = par_t * CH

        def vec(j, pm):
            cv = colbuf[pl.ds(boff + j * 16, 16)]
            rv = rowbuf[pl.ds(boff + j * 16, 16)]
            lv = cv - c0
            m = plsc.bitcast(lv, jnp.uint32) < jnp.uint32(RANGE)
            packed = jnp.where(m, rv * PACK + lv, jnp.int32(INVALID))
            packed = jnp.sort(packed)
            stage[pl.ds(pm, 16)] = packed
            return pm + plsc.all_reduce_population_count(m)[0]

        pm, hp = carry
        pm = lax.fori_loop(0, CH // 16, vec, pm, unroll=8)
        do_flush = (pm >= STG) & (hp + STG <= CAP)

        @pl.when(do_flush)
        def _():
            pltpu.sync_copy(
                stage.at[pl.ds(0, STG)],
                eown.at[pl.ds(pl.multiple_of(wid * CAP + hp, STG), STG)])

            def mvv(t, _):
                stage[pl.ds(t * 16, 16)] = stage[pl.ds(STG + t * 16, 16)]
                return 0
            lax.fori_loop(0, (pm - STG + 15) // 16, mvv, 0)

        hp = jnp.where(do_flush, hp + STG, hp)
        pm = jnp.where(do_flush, pm - STG, pm)
        return (pm, hp)

    pm, hp = lax.fori_loop(0, NCH, chunk, (0, 0))

    do2 = (pm >= STG) & (hp + STG <= CAP)

    @pl.when(do2)
    def _():
        pltpu.sync_copy(
            stage.at[pl.ds(0, STG)],
            eown.at[pl.ds(pl.multiple_of(wid * CAP + hp, STG), STG)])

        def mvv2(t, _):
            stage[pl.ds(t * 16, 16)] = stage[pl.ds(STG + t * 16, 16)]
            return 0
        lax.fori_loop(0, (pm - STG + 15) // 16, mvv2, 0)

    hp = jnp.where(do2, hp + STG, hp)
    pm = jnp.where(do2, pm - STG, pm)

    def fin(j, _):
        base = j * 16
        keep = (iota + base) < pm
        cur = stage[pl.ds(base, 16)]
        stage[pl.ds(base, 16)] = jnp.where(keep, cur, sent16(base))
        return 0
    lax.fori_loop(0, STG // 16, fin, 0)

    def fill(i, hp2):
        @pl.when(hp2 < CAP)
        def _():
            pltpu.sync_copy(
                stage.at[pl.ds(0, STG)],
                eown.at[pl.ds(pl.multiple_of(wid * CAP + hp2, STG), STG)])

        @pl.when(i == 0)
        def _():
            def allsent(j, _):
                base = j * 16
                stage[pl.ds(base, 16)] = sent16(base)
                return 0
            lax.fori_loop(0, STG // 16, allsent, 0)

        return jnp.where(hp2 < CAP, hp2 + STG, hp2)
    lax.fori_loop(0, CAP // STG, fill, hp)

    def degchunk(i, _):
        pltpu.sync_copy(
            eown.at[pl.ds(pl.multiple_of(wid * CAP + i * CH_D, CH_D), CH_D)],
            colbuf.at[pl.ds(0, CH_D)])

        def grp(g, _):
            clv = colbuf[pl.ds(g * 16, 16)] & (PACK - 1)
            for ee in range(16):
                cl = clv[ee]
                w = degbuf[pl.ds(cl, 16)]
                degbuf[pl.ds(cl, 16)] = w + one0
            return 0
        lax.fori_loop(0, CH_D // 16, grp, 0)
        return 0
    lax.fori_loop(0, CAP // CH_D, degchunk, 0)

    pltpu.sync_copy(degbuf.at[pl.ds(0, RANGE)],
                    deg_out.at[pl.ds(pl.multiple_of(wid * RANGE, 64), RANGE)])


K_PIPE = 2
GRP = ECH * K_PIPE
NG = CAP // GRP
ACC_R = 3152


def _make_round(FB, PP, want_g, blocked_src):
    out_type = [jax.ShapeDtypeStruct((PP * N_P * FB,), jnp.float32)]
    if want_g:
        out_type.append(jax.ShapeDtypeStruct((PP * N_P * FB,), jnp.float32))

    @functools.partial(
        pl.kernel,
        out_type=tuple(out_type),
        mesh=_mesh,
        compiler_params=pltpu.CompilerParams(
            needs_layout_passes=False, use_tc_tiling_on_sc=False),
        scratch_types=(
            pltpu.VMEM((2 * GRP,), jnp.int32),
            pltpu.VMEM((2 * GRP,), jnp.int32),
            pltpu.VMEM((2 * GRP,), jnp.int32),
            pltpu.VMEM((2 * GRP, FB), jnp.float32),
            pltpu.VMEM((ACC_R * FB,), jnp.float32),
            pltpu.VMEM((RANGE,), jnp.float32),
            pltpu.SemaphoreType.DMA,
            pltpu.SemaphoreType.DMA,
            pltpu.SemaphoreType.DMA,
            pltpu.SemaphoreType.DMA,
        ),
    )
    def round_k(g_hbm, eown, dis_hbm, *rest):
        if want_g:
            h_out, g_out = rest[0], rest[1]
            scratch = rest[2:]
        else:
            h_out = rest[0]
            scratch = rest[1:]
        (ebuf, idxbuf, clbuf, msgbuf, accflat, disbuf,
         gsem0, gsem1, esem0, esem1) = scratch
        gsems = (gsem0, gsem1)
        esems = (esem0, esem1)
        wid = _wid()
        c0 = wid * RANGE
        inf16 = jnp.full((16,), jnp.inf, jnp.float32)

        pltpu.sync_copy(dis_hbm.at[pl.ds(pl.multiple_of(c0, 64), RANGE)],
                        disbuf)

        for pass_ in range(PP):
            def eload(gi, par):
                off = par * GRP
                pltpu.async_copy(
                    eown.at[pl.ds(pl.multiple_of(wid * CAP + gi * GRP, GRP),
                                  GRP)],
                    ebuf.at[pl.ds(off, GRP)], esems[par])

            def launch(gi, par):
                off = par * GRP
                pltpu.make_async_copy(
                    eown.at[pl.ds(pl.multiple_of(wid * CAP + gi * GRP, GRP),
                                  GRP)],
                    ebuf.at[pl.ds(off, GRP)], esems[par]).wait()

                def mkidx(v, _):
                    ev = ebuf[pl.ds(off + v * 16, 16)]
                    rv = lax.shift_right_logical(ev, 12)
                    clbuf[pl.ds(off + v * 16, 16)] = (ev & (PACK - 1)) * FB
                    if blocked_src:
                        idxbuf[pl.ds(off + v * 16, 16)] = rv + pass_ * N_P
                    else:
                        idxbuf[pl.ds(off + v * 16, 16)] = rv * PP + pass_
                    return 0
                lax.fori_loop(0, GRP // 16, mkidx, 0)
                for k in range(K_PIPE):
                    pltpu.async_copy(
                        g_hbm.at[idxbuf.at[pl.ds(off + k * ECH, ECH)]],
                        msgbuf.at[pl.ds(off + k * ECH, ECH)],
                        gsems[par])

            def drain_process(par):
                off = par * GRP
                for k in range(K_PIPE):
                    pltpu.make_async_copy(
                        g_hbm.at[idxbuf.at[pl.ds(off + k * ECH, ECH)]],
                        msgbuf.at[pl.ds(off + k * ECH, ECH)],
                        gsems[par]).wait()

                def grp(g, _):
                    clv = clbuf[pl.ds(off + g * 16, 16)]
                    for ee in range(16):
                        base = clv[ee]
                        r = off + g * 16 + ee
                        for v in range(FB // 16):
                            a = accflat[pl.ds(base + v * 16, 16)]
                            mv = msgbuf[r, pl.ds(v * 16, 16)]
                            accflat[pl.ds(base + v * 16, 16)] = \
                                jnp.minimum(a, mv)
                    return 0
                lax.fori_loop(0, GRP // 16, grp, 0, unroll=2)

            def zacc(j, _):
                accflat[pl.ds(j * 16, 16)] = inf16
                return 0
            lax.fori_loop(0, ACC_R * FB // 16, zacc, 0)

            eload(0, 0)
            launch(0, 0)
            eload(1, 1)

            def body(g, _):
                par_t = g & 1

                @pl.when((par_t == 0) & (g + 1 < NG))
                def _():
                    launch(g + 1, 1)

                @pl.when((par_t == 1) & (g + 1 < NG))
                def _():
                    launch(g + 1, 0)

                @pl.when((par_t == 0) & (g + 2 < NG))
                def _():
                    eload(g + 2, 0)

                @pl.when((par_t == 1) & (g + 2 < NG))
                def _():
                    eload(g + 2, 1)

                @pl.when(par_t == 0)
                def _():
                    drain_process(0)

                @pl.when(par_t == 1)
                def _():
                    drain_process(1)
                return 0
            lax.fori_loop(0, NG, body, 0)

            def hrow(r16, _):
                dvec = disbuf[pl.ds(r16 * 16, 16)]
                for rr in range(16):
                    dv = dvec[rr]
                    off = (r16 * 16 + rr) * FB
                    for v in range(FB // 16):
                        a = accflat[pl.ds(off + v * 16, 16)]
                        a = jnp.where(a == jnp.inf, 0.0, a) * dv
                        accflat[pl.ds(off + v * 16, 16)] = a
                return 0
            lax.fori_loop(0, RANGE // 16, hrow, 0)
            pltpu.sync_copy(
                accflat.at[pl.ds(0, RANGE * FB)],
                h_out.at[pl.ds(pl.multiple_of(
                    pass_ * N_P * FB + c0 * FB, 256), RANGE * FB)])

            if want_g:
                def grow(r16, _):
                    dvec = disbuf[pl.ds(r16 * 16, 16)]
                    for rr in range(16):
                        dv = dvec[rr]
                        off = (r16 * 16 + rr) * FB
                        for v in range(FB // 16):
                            a = accflat[pl.ds(off + v * 16, 16)]
                            accflat[pl.ds(off + v * 16, 16)] = a * dv
                    return 0
                lax.fori_loop(0, RANGE // 16, grow, 0)
                pltpu.sync_copy(
                    accflat.at[pl.ds(0, RANGE * FB)],
                    g_out.at[pl.ds(pl.multiple_of(
                        pass_ * N_P * FB + c0 * FB, 256), RANGE * FB)])

    return round_k


_round16 = _make_round(16, 1, True, False)
_round16_h = _make_round(16, 1, False, False)
_round128 = _make_round(32, 4, True, False)
_round128_h = _make_round(32, 4, False, True)


def _prep_body(deg_ref, x_ref, dis_ref, gx_ref):
    deg = deg_ref[...]
    dis = jnp.where(deg > 0, lax.rsqrt(deg), 0.0)
    dis_ref[...] = dis
    gx_ref[...] = x_ref[...] * dis[:, None]


def _prep(deg, x_p):
    return pl.pallas_call(
        _prep_body,
        grid=(N_P // BN,),
        in_specs=[pl.BlockSpec((BN,), lambda i: (i,)),
                  pl.BlockSpec((BN, 16), lambda i: (i, 0))],
        out_specs=[pl.BlockSpec((BN,), lambda i: (i,)),
                   pl.BlockSpec((BN, 16), lambda i: (i, 0))],
        out_shape=[jax.ShapeDtypeStruct((N_P,), jnp.float32),
                   jax.ShapeDtypeStruct((N_P, 16), jnp.float32)],
    )(deg, x_p)


def _dense1_body(x_ref, h1_ref, h2_ref, dis_ref, w0_ref, w1_ref, w2_ref,
                 b_ref, h_ref, gh_ref):
    acc = jnp.dot(x_ref[...], w0_ref[...], preferred_element_type=jnp.float32)
    acc += jnp.dot(h1_ref[...], w1_ref[...], preferred_element_type=jnp.float32)
    acc += jnp.dot(h2_ref[...], w2_ref[...], preferred_element_type=jnp.float32)
    acc += b_ref[...][None, :]
    h = jnp.maximum(acc, 0.0)
    h_ref[...] = h
    gh_ref[...] = h * dis_ref[...][:, None]


def _dense1(x_p, h1, h2, dis, w0, w1, w2, b):
    in16 = pl.BlockSpec((BN, 16), lambda i: (i, 0))
    w_spec = pl.BlockSpec((16, D_HID), lambda i: (0, 0))
    return pl.pallas_call(
        _dense1_body,
        grid=(N_P // BN,),
        in_specs=[in16, in16, in16, pl.BlockSpec((BN,), lambda i: (i,)),
                  w_spec, w_spec, w_spec,
                  pl.BlockSpec((D_HID,), lambda i: (0,))],
        out_specs=[pl.BlockSpec((BN, D_HID), lambda i: (i, 0)),
                   pl.BlockSpec((BN, D_HID), lambda i: (i, 0))],
        out_shape=[jax.ShapeDtypeStruct((N_P, D_HID), jnp.float32),
                   jax.ShapeDtypeStruct((N_P, D_HID), jnp.float32)],
    )(x_p, h1, h2, dis, w0, w1, w2, b)


def _dense2_body(h_ref, *refs):
    h21 = refs[0:4]
    h22 = refs[4:8]
    w0_ref = refs[8]
    w21 = refs[9:13]
    w22 = refs[13:17]
    b_ref = refs[17]
    o_ref = refs[18]
    acc = jnp.dot(h_ref[...], w0_ref[...], preferred_element_type=jnp.float32)
    for p in range(4):
        acc += jnp.dot(h21[p][0], w21[p][...],
                       preferred_element_type=jnp.float32)
        acc += jnp.dot(h22[p][0], w22[p][...],
                       preferred_element_type=jnp.float32)
    acc += b_ref[...][None, :]
    m = jnp.max(acc, axis=1, keepdims=True)
    lse = jnp.log(jnp.sum(jnp.exp(acc - m), axis=1, keepdims=True)) + m
    o_ref[...] = acc - lse


def _dense2(h, h21, h22, w0, w21s, w22s, b):
    in_h = pl.BlockSpec((BN, D_HID), lambda i: (i, 0))

    def pspec(p):
        return pl.BlockSpec((1, BN, 32), lambda i, p=p: (p, i, 0))

    wp_spec = pl.BlockSpec((32, D_OUT), lambda i: (0, 0))
    return pl.pallas_call(
        _dense2_body,
        grid=(N_P // BN,),
        in_specs=([in_h] + [pspec(p) for p in range(4)] * 2
                  + [pl.BlockSpec((D_HID, D_OUT), lambda i: (0, 0))]
                  + [wp_spec] * 8
                  + [pl.BlockSpec((D_OUT,), lambda i: (0,))]),
        out_specs=pl.BlockSpec((BN, D_OUT), lambda i: (i, 0)),
        out_shape=jax.ShapeDtypeStruct((N_P, D_OUT), jnp.float32),
    )(h, h21, h21, h21, h21, h22, h22, h22, h22,
      w0, *w21s, *w22s, b)


def kernel(x, edge_index, W1_0, W1_1, W1_2, b1, W2_0, W2_1, W2_2, b2):
    row = edge_index[0].astype(jnp.int32)
    col = edge_index[1].astype(jnp.int32)
    x_p = jnp.pad(x, ((0, N_P - N), (0, 16 - D_IN)))
    w1_0 = jnp.pad(W1_0, ((0, 16 - D_IN), (0, 0)))
    w1_1 = jnp.pad(W1_1, ((0, 16 - D_IN), (0, 0)))
    w1_2 = jnp.pad(W1_2, ((0, 16 - D_IN), (0, 0)))
    w21s = [W2_1[32 * p:32 * (p + 1)] for p in range(4)]
    w22s = [W2_2[32 * p:32 * (p + 1)] for p in range(4)]

    eown, deg = _phase_a(row, col)
    dis, gx = _prep(deg, x_p)

    h11_f, g11_f = _round16(gx, eown, dis)
    h11 = h11_f.reshape(N_P, 16)
    g11 = g11_f.reshape(N_P, 16)
    (h12_f,) = _round16_h(g11, eown, dis)
    h12 = h12_f.reshape(N_P, 16)
    h, gh = _dense1(x_p, h11, h12, dis, w1_0, w1_1, w1_2, b1)

    h21_f, g21_f = _round128(gh.reshape(N_P * 4, 32), eown, dis)
    h21 = h21_f.reshape(4, N_P, 32)
    (h22_f,) = _round128_h(g21_f.reshape(N_P * 4, 32), eown, dis)
    h22 = h22_f.reshape(4, N_P, 32)

    out = _dense2(h, h21, h22, W2_0, w21s, w22s, b2)
    return out[:N]


# speedup vs baseline: 10.8883x; 1.0025x over previous
import functools

import jax
import jax.numpy as jnp
from jax import lax
from jax.experimental import pallas as pl
from jax.experimental.pallas import tpu as pltpu
from jax.experimental.pallas import tpu_sc as plsc

N = 100000
E = 3200000
D_IN = 11
D_HID = 128
D_OUT = 2

NW = 32
RANGE = 3136
N_P = NW * RANGE
CAP = 102400
STG = 2048
CH = 4000
CH_D = 2048
ACC_ROWS = 3200
SENT_CL = RANGE
ECH = 128

BN = 1024

_mesh = plsc.VectorSubcoreMesh(
    core_axis_name="c", subcore_axis_name="s", num_cores=2, num_subcores=16)


def _wid():
    return lax.axis_index("s") * 2 + lax.axis_index("c")


PACK = 4096
INVALID = 0x7FFFFFFF


@functools.partial(
    pl.kernel,
    out_type=(
        jax.ShapeDtypeStruct((NW * CAP,), jnp.int32),
        jax.ShapeDtypeStruct((N_P,), jnp.float32),
    ),
    mesh=_mesh,
    compiler_params=pltpu.CompilerParams(
        needs_layout_passes=False, use_tc_tiling_on_sc=False),
    scratch_types=(
        pltpu.VMEM((2 * CH,), jnp.int32),
        pltpu.VMEM((2 * CH,), jnp.int32),
        pltpu.VMEM((2 * STG + 16,), jnp.int32),
        pltpu.VMEM((ACC_ROWS + 16,), jnp.float32),
        pltpu.SemaphoreType.DMA,
        pltpu.SemaphoreType.DMA,
    ),
)
def _phase_a(row_hbm, col_hbm, eown, deg_out,
             colbuf, rowbuf, stage, degbuf, asem0, asem1):
    wid = _wid()
    c0 = wid * RANGE
    iota = lax.broadcasted_iota(jnp.int32, (16,), 0)
    zero16 = jnp.zeros((16,), jnp.float32)
    one0 = jnp.where(iota == 0, 1.0, 0.0).astype(jnp.float32)
    asems = (asem0, asem1)

    def sent16(base):
        return ((iota * 521 + base + wid * 1031) & 32767) * PACK + SENT_CL

    def zdeg(j, _):
        degbuf[pl.ds(j * 16, 16)] = zero16
        return 0
    lax.fori_loop(0, (ACC_ROWS + 16) // 16, zdeg, 0)

    NCH = E // CH

    def load(i, par):
        off = par * CH
        pltpu.async_copy(col_hbm.at[pl.ds(pl.multiple_of(i * CH, 8), CH)],
                         colbuf.at[pl.ds(off, CH)], asems[par])
        pltpu.async_copy(row_hbm.at[pl.ds(pl.multiple_of(i * CH, 8), CH)],
                         rowbuf.at[pl.ds(off, CH)], asems[par])

    def drain(i, par):
        off = par * CH
        pltpu.make_async_copy(
            col_hbm.at[pl.ds(pl.multiple_of(i * CH, 8), CH)],
            colbuf.at[pl.ds(off, CH)], asems[par]).wait()
        pltpu.make_async_copy(
            row_hbm.at[pl.ds(pl.multiple_of(i * CH, 8), CH)],
            rowbuf.at[pl.ds(off, CH)], asems[par]).wait()

    load(0, 0)

    def chunk(i, carry):
        par_t = i & 1

        @pl.when((par_t == 0) & (i + 1 < NCH))
        def _():
            load(i + 1, 1)

        @pl.when((par_t == 1) & (i + 1 < NCH))
        def _():
            load(i + 1, 0)

        @pl.when(par_t == 0)
        def _():
            drain(i, 0)

        @pl.when(par_t == 1)
        def _():
            drain(i, 1)

        boff = par_t * CH

        def vec(j, pm):
            cv = colbuf[pl.ds(boff + j * 16, 16)]
            rv = rowbuf[pl.ds(boff + j * 16, 16)]
            lv = cv - c0
            m = plsc.bitcast(lv, jnp.uint32) < jnp.uint32(RANGE)
            packed = jnp.where(m, rv * PACK + lv, jnp.int32(INVALID))
            packed = jnp.sort(packed)
            stage[pl.ds(pm, 16)] = packed
            return pm + plsc.all_reduce_population_count(m)[0]

        pm, hp = carry
        pm = lax.fori_loop(0, CH // 16, vec, pm, unroll=8)
        do_flush = (pm >= STG) & (hp + STG <= CAP)

        @pl.when(do_flush)
        def _():
            pltpu.sync_copy(
                stage.at[pl.ds(0, STG)],
                eown.at[pl.ds(pl.multiple_of(wid * CAP + hp, STG), STG)])

            def mvv(t, _):
                stage[pl.ds(t * 16, 16)] = stage[pl.ds(STG + t * 16, 16)]
                return 0
            lax.fori_loop(0, (pm - STG + 15) // 16, mvv, 0)

        hp = jnp.where(do_flush, hp + STG, hp)
        pm = jnp.where(do_flush, pm - STG, pm)
        return (pm, hp)

    pm, hp = lax.fori_loop(0, NCH, chunk, (0, 0))

    do2 = (pm >= STG) & (hp + STG <= CAP)

    @pl.when(do2)
    def _():
        pltpu.sync_copy(
            stage.at[pl.ds(0, STG)],
            eown.at[pl.ds(pl.multiple_of(wid * CAP + hp, STG), STG)])

        def mvv2(t, _):
            stage[pl.ds(t * 16, 16)] = stage[pl.ds(STG + t * 16, 16)]
            return 0
        lax.fori_loop(0, (pm - STG + 15) // 16, mvv2, 0)

    hp = jnp.where(do2, hp + STG, hp)
    pm = jnp.where(do2, pm - STG, pm)

    def fin(j, _):
        base = j * 16
        keep = (iota + base) < pm
        cur = stage[pl.ds(base, 16)]
        stage[pl.ds(base, 16)] = jnp.where(keep, cur, sent16(base))
        return 0
    lax.fori_loop(0, STG // 16, fin, 0)

    def fill(i, hp2):
        @pl.when(hp2 < CAP)
        def _():
            pltpu.sync_copy(
                stage.at[pl.ds(0, STG)],
                eown.at[pl.ds(pl.multiple_of(wid * CAP + hp2, STG), STG)])

        @pl.when(i == 0)
        def _():
            def allsent(j, _):
                base = j * 16
                stage[pl.ds(base, 16)] = sent16(base)
                return 0
            lax.fori_loop(0, STG // 16, allsent, 0)

        return jnp.where(hp2 < CAP, hp2 + STG, hp2)
    lax.fori_loop(0, CAP // STG, fill, hp)

    def degchunk(i, _):
        pltpu.sync_copy(
            eown.at[pl.ds(pl.multiple_of(wid * CAP + i * CH_D, CH_D), CH_D)],
            colbuf.at[pl.ds(0, CH_D)])

        def grp(g, _):
            clv = colbuf[pl.ds(g * 16, 16)] & (PACK - 1)
            for ee in range(16):
                cl = clv[ee]
                w = degbuf[pl.ds(cl, 16)]
                degbuf[pl.ds(cl, 16)] = w + one0
            return 0
        lax.fori_loop(0, CH_D // 16, grp, 0)
        return 0
    lax.fori_loop(0, CAP // CH_D, degchunk, 0)

    pltpu.sync_copy(degbuf.at[pl.ds(0, RANGE)],
                    deg_out.at[pl.ds(pl.multiple_of(wid * RANGE, 64), RANGE)])


K_PIPE = 2
GRP = ECH * K_PIPE
NG = CAP // GRP
ACC_R = 3152


def _make_round(FB, PP, want_g, blocked_src):
    out_type = [jax.ShapeDtypeStruct((PP * N_P * FB,), jnp.float32)]
    if want_g:
        out_type.append(jax.ShapeDtypeStruct((PP * N_P * FB,), jnp.float32))

    @functools.partial(
        pl.kernel,
        out_type=tuple(out_type),
        mesh=_mesh,
        compiler_params=pltpu.CompilerParams(
            needs_layout_passes=False, use_tc_tiling_on_sc=False),
        scratch_types=(
            pltpu.VMEM((2 * GRP,), jnp.int32),
            pltpu.VMEM((2 * GRP,), jnp.int32),
            pltpu.VMEM((2 * GRP,), jnp.int32),
            pltpu.VMEM((2 * GRP, FB), jnp.float32),
            pltpu.VMEM((ACC_R * FB,), jnp.float32),
            pltpu.VMEM((RANGE,), jnp.float32),
            pltpu.SemaphoreType.DMA,
            pltpu.SemaphoreType.DMA,
            pltpu.SemaphoreType.DMA,
            pltpu.SemaphoreType.DMA,
        ),
    )
    def round_k(g_hbm, eown, dis_hbm, *rest):
        if want_g:
            h_out, g_out = rest[0], rest[1]
            scratch = rest[2:]
        else:
            h_out = rest[0]
            scratch = rest[1:]
        (ebuf, idxbuf, clbuf, msgbuf, accflat, disbuf,
         gsem0, gsem1, esem0, esem1) = scratch
        gsems = (gsem0, gsem1)
        esems = (esem0, esem1)
        wid = _wid()
        c0 = wid * RANGE
        inf16 = jnp.full((16,), jnp.inf, jnp.float32)

        pltpu.sync_copy(dis_hbm.at[pl.ds(pl.multiple_of(c0, 64), RANGE)],
                        disbuf)

        for pass_ in range(PP):
            def eload(gi, par):
                off = par * GRP
                pltpu.async_copy(
                    eown.at[pl.ds(pl.multiple_of(wid * CAP + gi * GRP, GRP),
                                  GRP)],
                    ebuf.at[pl.ds(off, GRP)], esems[par])

            def launch(gi, par):
                off = par * GRP
                pltpu.make_async_copy(
                    eown.at[pl.ds(pl.multiple_of(wid * CAP + gi * GRP, GRP),
                                  GRP)],
                    ebuf.at[pl.ds(off, GRP)], esems[par]).wait()

                def mkidx(v, _):
                    ev = ebuf[pl.ds(off + v * 16, 16)]
                    rv = lax.shift_right_logical(ev, 12)
                    clbuf[pl.ds(off + v * 16, 16)] = (ev & (PACK - 1)) * FB
                    if blocked_src:
                        idxbuf[pl.ds(off + v * 16, 16)] = rv + pass_ * N_P
                    else:
                        idxbuf[pl.ds(off + v * 16, 16)] = rv * PP + pass_
                    return 0
                lax.fori_loop(0, GRP // 16, mkidx, 0, unroll=4)
                for k in range(K_PIPE):
                    pltpu.async_copy(
                        g_hbm.at[idxbuf.at[pl.ds(off + k * ECH, ECH)]],
                        msgbuf.at[pl.ds(off + k * ECH, ECH)],
                        gsems[par])

            def drain_process(par):
                off = par * GRP
                for k in range(K_PIPE):
                    pltpu.make_async_copy(
                        g_hbm.at[idxbuf.at[pl.ds(off + k * ECH, ECH)]],
                        msgbuf.at[pl.ds(off + k * ECH, ECH)],
                        gsems[par]).wait()

                def grp(g, _):
                    clv = clbuf[pl.ds(off + g * 16, 16)]
                    for ee in range(16):
                        base = clv[ee]
                        r = off + g * 16 + ee
                        for v in range(FB // 16):
                            a = accflat[pl.ds(base + v * 16, 16)]
                            mv = msgbuf[r, pl.ds(v * 16, 16)]
                            accflat[pl.ds(base + v * 16, 16)] = \
                                jnp.minimum(a, mv)
                    return 0
                lax.fori_loop(0, GRP // 16, grp, 0, unroll=4)

            def zacc(j, _):
                accflat[pl.ds(j * 16, 16)] = inf16
                return 0
            lax.fori_loop(0, ACC_R * FB // 16, zacc, 0)

            eload(0, 0)
            launch(0, 0)
            eload(1, 1)

            def body(g, _):
                par_t = g & 1

                @pl.when((par_t == 0) & (g + 1 < NG))
                def _():
                    launch(g + 1, 1)

                @pl.when((par_t == 1) & (g + 1 < NG))
                def _():
                    launch(g + 1, 0)

                @pl.when((par_t == 0) & (g + 2 < NG))
                def _():
                    eload(g + 2, 0)

                @pl.when((par_t == 1) & (g + 2 < NG))
                def _():
                    eload(g + 2, 1)

                @pl.when(par_t == 0)
                def _():
                    drain_process(0)

                @pl.when(par_t == 1)
                def _():
                    drain_process(1)
                return 0
            lax.fori_loop(0, NG, body, 0)

            def hrow(r16, _):
                dvec = disbuf[pl.ds(r16 * 16, 16)]
                for rr in range(16):
                    dv = dvec[rr]
                    off = (r16 * 16 + rr) * FB
                    for v in range(FB // 16):
                        a = accflat[pl.ds(off + v * 16, 16)]
                        a = jnp.where(a == jnp.inf, 0.0, a) * dv
                        accflat[pl.ds(off + v * 16, 16)] = a
                return 0
            lax.fori_loop(0, RANGE // 16, hrow, 0)
            pltpu.sync_copy(
                accflat.at[pl.ds(0, RANGE * FB)],
                h_out.at[pl.ds(pl.multiple_of(
                    pass_ * N_P * FB + c0 * FB, 256), RANGE * FB)])

            if want_g:
                def grow(r16, _):
                    dvec = disbuf[pl.ds(r16 * 16, 16)]
                    for rr in range(16):
                        dv = dvec[rr]
                        off = (r16 * 16 + rr) * FB
                        for v in range(FB // 16):
                            a = accflat[pl.ds(off + v * 16, 16)]
                            accflat[pl.ds(off + v * 16, 16)] = a * dv
                    return 0
                lax.fori_loop(0, RANGE // 16, grow, 0)
                pltpu.sync_copy(
                    accflat.at[pl.ds(0, RANGE * FB)],
                    g_out.at[pl.ds(pl.multiple_of(
                        pass_ * N_P * FB + c0 * FB, 256), RANGE * FB)])

    return round_k


_round16 = _make_round(16, 1, True, False)
_round16_h = _make_round(16, 1, False, False)
_round128 = _make_round(32, 4, True, False)
_round128_h = _make_round(32, 4, False, True)


def _prep_body(deg_ref, x_ref, dis_ref, gx_ref):
    deg = deg_ref[...]
    dis = jnp.where(deg > 0, lax.rsqrt(deg), 0.0)
    dis_ref[...] = dis
    gx_ref[...] = x_ref[...] * dis[:, None]


def _prep(deg, x_p):
    return pl.pallas_call(
        _prep_body,
        grid=(N_P // BN,),
        in_specs=[pl.BlockSpec((BN,), lambda i: (i,)),
                  pl.BlockSpec((BN, 16), lambda i: (i, 0))],
        out_specs=[pl.BlockSpec((BN,), lambda i: (i,)),
                   pl.BlockSpec((BN, 16), lambda i: (i, 0))],
        out_shape=[jax.ShapeDtypeStruct((N_P,), jnp.float32),
                   jax.ShapeDtypeStruct((N_P, 16), jnp.float32)],
    )(deg, x_p)


def _dense1_body(x_ref, h1_ref, h2_ref, dis_ref, w0_ref, w1_ref, w2_ref,
                 b_ref, h_ref, gh_ref):
    acc = jnp.dot(x_ref[...], w0_ref[...], preferred_element_type=jnp.float32)
    acc += jnp.dot(h1_ref[...], w1_ref[...], preferred_element_type=jnp.float32)
    acc += jnp.dot(h2_ref[...], w2_ref[...], preferred_element_type=jnp.float32)
    acc += b_ref[...][None, :]
    h = jnp.maximum(acc, 0.0)
    h_ref[...] = h
    gh_ref[...] = h * dis_ref[...][:, None]


def _dense1(x_p, h1, h2, dis, w0, w1, w2, b):
    in16 = pl.BlockSpec((BN, 16), lambda i: (i, 0))
    w_spec = pl.BlockSpec((16, D_HID), lambda i: (0, 0))
    return pl.pallas_call(
        _dense1_body,
        grid=(N_P // BN,),
        in_specs=[in16, in16, in16, pl.BlockSpec((BN,), lambda i: (i,)),
                  w_spec, w_spec, w_spec,
                  pl.BlockSpec((D_HID,), lambda i: (0,))],
        out_specs=[pl.BlockSpec((BN, D_HID), lambda i: (i, 0)),
                   pl.BlockSpec((BN, D_HID), lambda i: (i, 0))],
        out_shape=[jax.ShapeDtypeStruct((N_P, D_HID), jnp.float32),
                   jax.ShapeDtypeStruct((N_P, D_HID), jnp.float32)],
    )(x_p, h1, h2, dis, w0, w1, w2, b)


def _dense2_body(h_ref, *refs):
    h21 = refs[0:4]
    h22 = refs[4:8]
    w0_ref = refs[8]
    w21 = refs[9:13]
    w22 = refs[13:17]
    b_ref = refs[17]
    o_ref = refs[18]
    acc = jnp.dot(h_ref[...], w0_ref[...], preferred_element_type=jnp.float32)
    for p in range(4):
        acc += jnp.dot(h21[p][0], w21[p][...],
                       preferred_element_type=jnp.float32)
        acc += jnp.dot(h22[p][0], w22[p][...],
                       preferred_element_type=jnp.float32)
    acc += b_ref[...][None, :]
    m = jnp.max(acc, axis=1, keepdims=True)
    lse = jnp.log(jnp.sum(jnp.exp(acc - m), axis=1, keepdims=True)) + m
    o_ref[...] = acc - lse


def _dense2(h, h21, h22, w0, w21s, w22s, b):
    in_h = pl.BlockSpec((BN, D_HID), lambda i: (i, 0))

    def pspec(p):
        return pl.BlockSpec((1, BN, 32), lambda i, p=p: (p, i, 0))

    wp_spec = pl.BlockSpec((32, D_OUT), lambda i: (0, 0))
    return pl.pallas_call(
        _dense2_body,
        grid=(N_P // BN,),
        in_specs=([in_h] + [pspec(p) for p in range(4)] * 2
                  + [pl.BlockSpec((D_HID, D_OUT), lambda i: (0, 0))]
                  + [wp_spec] * 8
                  + [pl.BlockSpec((D_OUT,), lambda i: (0,))]),
        out_specs=pl.BlockSpec((BN, D_OUT), lambda i: (i, 0)),
        out_shape=jax.ShapeDtypeStruct((N_P, D_OUT), jnp.float32),
    )(h, h21, h21, h21, h21, h22, h22, h22, h22,
      w0, *w21s, *w22s, b)


def kernel(x, edge_index, W1_0, W1_1, W1_2, b1, W2_0, W2_1, W2_2, b2):
    row = edge_index[0].astype(jnp.int32)
    col = edge_index[1].astype(jnp.int32)
    x_p = jnp.pad(x, ((0, N_P - N), (0, 16 - D_IN)))
    w1_0 = jnp.pad(W1_0, ((0, 16 - D_IN), (0, 0)))
    w1_1 = jnp.pad(W1_1, ((0, 16 - D_IN), (0, 0)))
    w1_2 = jnp.pad(W1_2, ((0, 16 - D_IN), (0, 0)))
    w21s = [W2_1[32 * p:32 * (p + 1)] for p in range(4)]
    w22s = [W2_2[32 * p:32 * (p + 1)] for p in range(4)]

    eown, deg = _phase_a(row, col)
    dis, gx = _prep(deg, x_p)

    h11_f, g11_f = _round16(gx, eown, dis)
    h11 = h11_f.reshape(N_P, 16)
    g11 = g11_f.reshape(N_P, 16)
    (h12_f,) = _round16_h(g11, eown, dis)
    h12 = h12_f.reshape(N_P, 16)
    h, gh = _dense1(x_p, h11, h12, dis, w1_0, w1_1, w1_2, b1)

    h21_f, g21_f = _round128(gh.reshape(N_P * 4, 32), eown, dis)
    h21 = h21_f.reshape(4, N_P, 32)
    (h22_f,) = _round128_h(g21_f.reshape(N_P * 4, 32), eown, dis)
    h22 = h22_f.reshape(4, N_P, 32)

    out = _dense2(h, h21, h22, W2_0, w21s, w22s, b2)
    return out[:N]
